# baseline (speedup 1.0000x reference)
import functools

import jax
import jax.numpy as jnp
from jax import lax
from jax.experimental import pallas as pl
from jax.experimental.pallas import tpu as pltpu
from jax.experimental.pallas import tpu_sc as plsc

N_U = 10000
N_I = 5000
DIM = 64
TEMP = 0.2
LAMBDA_1 = 0.2
LAMBDA_2 = 1e-07
SVD_Q = 32
BATCH = 4096

BM = 1000
GRID_U = N_U // BM
F32 = jnp.float32
_HI = jax.lax.Precision.HIGHEST


def _dot(a, b):
    return jax.lax.dot_general(a, b, (((1,), (0,)), ((), ())),
                               precision=_HI, preferred_element_type=F32)


def _dot_t0(a, b):
    return jax.lax.dot_general(a, b, (((0,), (0,)), ((), ())),
                               precision=_HI, preferred_element_type=F32)


def _dot_t1(a, b):
    return jax.lax.dot_general(a, b, (((1,), (1,)), ((), ())),
                               precision=_HI, preferred_element_type=F32)


def _p1_body(a_ref, gt_ref, y_ref, c_ref):
    y = _dot_t1(a_ref[...], gt_ref[...])
    y_ref[...] = y

    @pl.when(pl.program_id(0) == 0)
    def _():
        c_ref[...] = jnp.zeros_like(c_ref)
    c_ref[...] += _dot_t0(y, y)


def _pass_fwd_gram(adj, gt):
    kq = gt.shape[0]
    return pl.pallas_call(
        _p1_body,
        grid=(GRID_U,),
        in_specs=[pl.BlockSpec((BM, N_I), lambda i: (i, 0)),
                  pl.BlockSpec((kq, N_I), lambda i: (0, 0))],
        out_specs=[pl.BlockSpec((BM, kq), lambda i: (i, 0)),
                   pl.BlockSpec((kq, kq), lambda i: (0, 0))],
        out_shape=[jax.ShapeDtypeStruct((N_U, kq), F32),
                   jax.ShapeDtypeStruct((kq, kq), F32)],
    )(adj, gt)


def _p2_body(a_ref, w_ref, x_ref, o_ref):
    q = _dot_t1(w_ref[...], x_ref[...])

    @pl.when(pl.program_id(0) == 0)
    def _():
        o_ref[...] = jnp.zeros_like(o_ref)
    o_ref[...] += _dot_t0(q, a_ref[...])


def _pass_transpose(adj, w, x):
    kq = w.shape[1]
    return pl.pallas_call(
        _p2_body,
        grid=(GRID_U,),
        in_specs=[pl.BlockSpec((BM, N_I), lambda i: (i, 0)),
                  pl.BlockSpec((BM, kq), lambda i: (i, 0)),
                  pl.BlockSpec((kq, kq), lambda i: (0, 0))],
        out_specs=pl.BlockSpec((kq, N_I), lambda i: (0, 0)),
        out_shape=jax.ShapeDtypeStruct((kq, N_I), F32),
    )(adj, w, x)


def _gram_t_body(wt_ref, c_ref):
    wt = wt_ref[...]
    c_ref[...] = _dot_t1(wt, wt)


def _gram_t(wt):
    kq = wt.shape[0]
    return pl.pallas_call(
        _gram_t_body,
        out_shape=jax.ShapeDtypeStruct((kq, kq), F32),
    )(wt)


def _cholinv_math(C):
    q = SVD_Q
    ri = jax.lax.broadcasted_iota(jnp.int32, (q, q), 0)
    ci = jax.lax.broadcasted_iota(jnp.int32, (q, q), 1)
    ri1 = jax.lax.broadcasted_iota(jnp.int32, (q, 1), 0)
    ci1 = jax.lax.broadcasted_iota(jnp.int32, (1, q), 1)

    def chol_step(j, carry):
        L, Ck = carry
        dj = jnp.sum(jnp.where((ri == j) & (ci == j), Ck, 0.0))
        inv_s = jax.lax.rsqrt(dj)
        col = jnp.sum(jnp.where(ci == j, Ck, 0.0), axis=1, keepdims=True)
        col = jnp.where(ri1 >= j, col, 0.0) * inv_s
        row = jnp.sum(jnp.where(ri == j, Ck, 0.0), axis=0, keepdims=True)
        row = jnp.where(ci1 >= j, row, 0.0) * inv_s
        L = L + jnp.where(ci == j, col, 0.0)
        Ck = Ck - col * row
        return L, Ck

    L, _ = jax.lax.fori_loop(0, q, chol_step, (jnp.zeros_like(C), C))

    def inv_step(i, X):
        lrow = jnp.sum(jnp.where(ri == i, L, 0.0), axis=0, keepdims=True)
        dii = jnp.sum(jnp.where(ci1 == i, lrow, 0.0))
        lrow = jnp.where(ci1 < i, lrow, 0.0)
        prod = _dot(lrow, X)
        xrow = (jnp.where(ci1 == i, 1.0, 0.0) - prod) / dii
        return X + jnp.where(ri == i, xrow, 0.0)

    return jax.lax.fori_loop(0, q, inv_step, jnp.zeros_like(C))


def _cholinv_body(c_ref, x_ref):
    x_ref[...] = _cholinv_math(c_ref[...])


def _cholinv(c):
    kq = c.shape[0]
    return pl.pallas_call(
        _cholinv_body,
        out_shape=jax.ShapeDtypeStruct((kq, kq), F32),
    )(c)


def _cholapply_body(c_ref, wt_ref, zt_ref):
    zt_ref[...] = _dot(_cholinv_math(c_ref[...]), wt_ref[...])


def _cholapply(c, wt):
    kq, n = wt.shape
    return pl.pallas_call(
        _cholapply_body,
        out_shape=jax.ShapeDtypeStruct((kq, n), F32),
    )(c, wt)


def _p6_body(a_ref, w_ref, x_ref, eu0_ref, ei0t_ref,
             q_ref, b_ref, zu1_ref, zi1t_ref, reg_ref):
    qb = _dot_t1(w_ref[...], x_ref[...])
    q_ref[...] = qb
    zu1_ref[...] = _dot_t1(a_ref[...], ei0t_ref[...])

    @pl.when(pl.program_id(0) == 0)
    def _():
        b_ref[...] = jnp.zeros_like(b_ref)
        zi1t_ref[...] = jnp.zeros_like(zi1t_ref)
        reg_ref[...] = jnp.reshape(
            jnp.sum(ei0t_ref[...] * ei0t_ref[...]), (1, 1))

    b_ref[...] += _dot_t0(qb, a_ref[...])
    zi1t_ref[...] += _dot_t0(eu0_ref[...], a_ref[...])
    reg_ref[...] += jnp.reshape(jnp.sum(eu0_ref[...] * eu0_ref[...]), (1, 1))


def _pass_six(adj, w, x, eu0, ei0t):
    return pl.pallas_call(
        _p6_body,
        grid=(GRID_U,),
        in_specs=[pl.BlockSpec((BM, N_I), lambda i: (i, 0)),
                  pl.BlockSpec((BM, SVD_Q), lambda i: (i, 0)),
                  pl.BlockSpec((SVD_Q, SVD_Q), lambda i: (0, 0)),
                  pl.BlockSpec((BM, DIM), lambda i: (i, 0)),
                  pl.BlockSpec((DIM, N_I), lambda i: (0, 0))],
        out_specs=[pl.BlockSpec((BM, SVD_Q), lambda i: (i, 0)),
                   pl.BlockSpec((SVD_Q, N_I), lambda i: (0, 0)),
                   pl.BlockSpec((BM, DIM), lambda i: (i, 0)),
                   pl.BlockSpec((DIM, N_I), lambda i: (0, 0)),
                   pl.BlockSpec((1, 1), lambda i: (0, 0))],
        out_shape=[jax.ShapeDtypeStruct((N_U, SVD_Q), F32),
                   jax.ShapeDtypeStruct((SVD_Q, N_I), F32),
                   jax.ShapeDtypeStruct((N_U, DIM), F32),
                   jax.ShapeDtypeStruct((DIM, N_I), F32),
                   jax.ShapeDtypeStruct((1, 1), F32)],
    )(adj, w, x, eu0, ei0t)


def _p7_body(a_ref, zu1_ref, zi1t_ref, eu0_ref, ei0t_ref, eu_ref, eit_ref):
    eu_ref[...] = (eu0_ref[...] + zu1_ref[...]
                   + _dot_t1(a_ref[...], zi1t_ref[...]))

    @pl.when(pl.program_id(0) == 0)
    def _():
        eit_ref[...] = ei0t_ref[...] + zi1t_ref[...]
    eit_ref[...] += _dot_t0(zu1_ref[...], a_ref[...])


def _pass_seven(adj, zu1, zi1t, eu0, ei0t):
    return pl.pallas_call(
        _p7_body,
        grid=(GRID_U,),
        in_specs=[pl.BlockSpec((BM, N_I), lambda i: (i, 0)),
                  pl.BlockSpec((BM, DIM), lambda i: (i, 0)),
                  pl.BlockSpec((DIM, N_I), lambda i: (0, 0)),
                  pl.BlockSpec((BM, DIM), lambda i: (i, 0)),
                  pl.BlockSpec((DIM, N_I), lambda i: (0, 0))],
        out_specs=[pl.BlockSpec((BM, DIM), lambda i: (i, 0)),
                   pl.BlockSpec((DIM, N_I), lambda i: (0, 0))],
        out_shape=[jax.ShapeDtypeStruct((N_U, DIM), F32),
                   jax.ShapeDtypeStruct((DIM, N_I), F32)],
    )(adj, zu1, zi1t, eu0, ei0t)


def _t12_body(b_ref, ei0t_ref, zi1t_ref, q_ref, eu0_ref, zu1_ref,
              t1_ref, t2t_ref):
    @pl.when(pl.program_id(0) == 0)
    def _():
        t1_ref[...] = _dot_t1(b_ref[...], ei0t_ref[...] + zi1t_ref[...])
        t2t_ref[...] = jnp.zeros_like(t2t_ref)
    t2t_ref[...] += _dot_t0(eu0_ref[...] + zu1_ref[...], q_ref[...])


def _t12(b, ei0t, zi1t, q, eu0, zu1):
    return pl.pallas_call(
        _t12_body,
        grid=(GRID_U,),
        in_specs=[pl.BlockSpec((SVD_Q, N_I), lambda i: (0, 0)),
                  pl.BlockSpec((DIM, N_I), lambda i: (0, 0)),
                  pl.BlockSpec((DIM, N_I), lambda i: (0, 0)),
                  pl.BlockSpec((BM, SVD_Q), lambda i: (i, 0)),
                  pl.BlockSpec((BM, DIM), lambda i: (i, 0)),
                  pl.BlockSpec((BM, DIM), lambda i: (i, 0))],
        out_specs=[pl.BlockSpec((SVD_Q, DIM), lambda i: (0, 0)),
                   pl.BlockSpec((DIM, SVD_Q), lambda i: (0, 0))],
        out_shape=[jax.ShapeDtypeStruct((SVD_Q, DIM), F32),
                   jax.ShapeDtypeStruct((DIM, SVD_Q), F32)],
    )(b, ei0t, zi1t, q, eu0, zu1)


def _gu_body(q_ref, t1_ref, eu0_ref, eu_ref, out_ref):
    gu = eu0_ref[...] + _dot(q_ref[...], t1_ref[...])
    out_ref[...] = jnp.concatenate([gu, eu_ref[...]], axis=1)


def _gu(q, t1, eu0, eu):
    return pl.pallas_call(
        _gu_body,
        grid=(GRID_U,),
        in_specs=[pl.BlockSpec((BM, SVD_Q), lambda i: (i, 0)),
                  pl.BlockSpec((SVD_Q, DIM), lambda i: (0, 0)),
                  pl.BlockSpec((BM, DIM), lambda i: (i, 0)),
                  pl.BlockSpec((BM, DIM), lambda i: (i, 0))],
        out_specs=pl.BlockSpec((BM, 2 * DIM), lambda i: (i, 0)),
        out_shape=jax.ShapeDtypeStruct((N_U, 2 * DIM), F32),
    )(q, t1, eu0, eu)


def _gi_body(b_ref, t2t_ref, ei0t_ref, eit_ref, out_ref):
    git = ei0t_ref[...] + _dot(t2t_ref[...], b_ref[...])
    out_ref[...] = jnp.concatenate([git, eit_ref[...]], axis=0)


def _gi(b, t2t, ei0t, eit):
    return pl.pallas_call(
        _gi_body,
        out_shape=jax.ShapeDtypeStruct((2 * DIM, N_I), F32),
    )(b, t2t, ei0t, eit)


def _sc_gather_all(ue_u, ue_i, row_ids, col_ids, pos, neg):
    info = plsc.get_sparse_core_info()
    nc, ns = info.num_cores, info.num_subcores
    nw = nc * ns
    bpw = BATCH // nw
    mesh = plsc.VectorSubcoreMesh(core_axis_name="c", subcore_axis_name="s")
    out = jax.ShapeDtypeStruct((BATCH, 2 * DIM), F32)

    @functools.partial(
        pl.kernel, mesh=mesh,
        out_type=(out,) * 4,
        scratch_types=[pltpu.VMEM((bpw,), jnp.int32),
                       pltpu.VMEM((bpw, 2 * DIM), F32),
                       pltpu.SemaphoreType.DMA],
    )
    def k(tu_hbm, ti_hbm, rid_hbm, cid_hbm, pos_hbm, neg_hbm,
          o_ru, o_rc, o_rp, o_rn, idx_v, rows_v, sem):
        wid = lax.axis_index("s") * nc + lax.axis_index("c")
        base = wid * bpw

        def gather(idx_hbm, table_hbm, out_hbm):
            pltpu.sync_copy(idx_hbm.at[pl.ds(base, bpw)], idx_v)
            pltpu.async_copy(table_hbm.at[idx_v], rows_v, sem).wait()
            pltpu.sync_copy(rows_v, out_hbm.at[pl.ds(base, bpw)])

        gather(rid_hbm, tu_hbm, o_ru)
        gather(cid_hbm, ti_hbm, o_rc)
        gather(pos_hbm, ti_hbm, o_rp)
        gather(neg_hbm, ti_hbm, o_rn)

    return k(ue_u, ue_i, row_ids, col_ids, pos, neg)


def _lse_body(n_chunks, chunk, table_t, g_ref, e_ref, o_ref):
    g = g_ref[:, :DIM]
    s = jnp.zeros((g.shape[0], 1), F32)
    for kc in range(n_chunks):
        if table_t:
            logits = _dot(g, e_ref[:, kc * chunk:(kc + 1) * chunk])
        else:
            logits = _dot_t1(g, e_ref[kc * chunk:(kc + 1) * chunk, :])
        s = s + jnp.sum(jnp.exp(logits * (1.0 / TEMP)), axis=1, keepdims=True)

    @pl.when(pl.program_id(0) == 0)
    def _():
        o_ref[...] = jnp.zeros_like(o_ref)
    o_ref[...] += jnp.reshape(jnp.sum(jnp.log(s + 1e-08)), (1, 1))


def _log_partition(rows, table, chunk, table_t):
    n = table.shape[1] if table_t else table.shape[0]
    bb = 512
    body = functools.partial(_lse_body, n // chunk, chunk, table_t)
    return pl.pallas_call(
        body,
        grid=(BATCH // bb,),
        in_specs=[pl.BlockSpec((bb, 2 * DIM), lambda i: (i, 0)),
                  pl.BlockSpec(table.shape, lambda i: (0, 0))],
        out_specs=pl.BlockSpec((1, 1), lambda i: (0, 0)),
        out_shape=jax.ShapeDtypeStruct((1, 1), F32),
    )(rows, table)


def _final_body(nlu_ref, nli_ref, reg_ref, ru_ref, rc_ref, rp_ref, rn_ref,
                loss_ref, oth_ref):
    inv_b = 1.0 / BATCH
    gur, eur = ru_ref[:, :DIM], ru_ref[:, DIM:]
    gic, eic = rc_ref[:, :DIM], rc_ref[:, DIM:]
    eip, ein = rp_ref[:, DIM:], rn_ref[:, DIM:]

    neg_score = (nlu_ref[0, 0] + nli_ref[0, 0]) * inv_b
    pu = jnp.sum(gur * eur, axis=1, keepdims=True) * (1.0 / TEMP)
    pi = jnp.sum(gic * eic, axis=1, keepdims=True) * (1.0 / TEMP)
    pos_score = (jnp.sum(jnp.clip(pu, -5.0, 5.0)) +
                 jnp.sum(jnp.clip(pi, -5.0, 5.0))) * inv_b
    loss_cl = -pos_score + neg_score

    d = jnp.sum(eur * eip, axis=1, keepdims=True) - \
        jnp.sum(eur * ein, axis=1, keepdims=True)
    loss_bpr = jnp.sum(jnp.log(1.0 + jnp.exp(-d))) * inv_b

    loss = loss_bpr + LAMBDA_1 * loss_cl + LAMBDA_2 * reg_ref[0, 0]
    loss_ref[...] = jnp.reshape(loss, (1, 1))
    oth_ref[...] = jnp.concatenate(
        [jnp.full((1, 1), loss_bpr, F32),
         jnp.full((1, 1), LAMBDA_1 * loss_cl, F32)], axis=1)


def _final(nlu, nli, reg, ru, rc, rp, rn):
    bspec = pl.BlockSpec((BATCH, 2 * DIM), lambda: (0, 0))
    sspec = pl.BlockSpec((1, 1), lambda: (0, 0))
    return pl.pallas_call(
        _final_body,
        in_specs=[sspec, sspec, sspec] + [bspec] * 4,
        out_specs=[sspec, pl.BlockSpec((1, 2), lambda: (0, 0))],
        out_shape=[jax.ShapeDtypeStruct((1, 1), F32),
                   jax.ShapeDtypeStruct((1, 2), F32)],
    )(nlu, nli, reg, ru, rc, rp, rn)


def kernel(adj, row_ids, col_ids, pos, neg, E_u_0, E_i_0):
    g0t = jax.random.normal(jax.random.key(42), (N_I, SVD_Q), dtype=F32).T
    ei0t = E_i_0.T

    y0, c0 = _pass_fwd_gram(adj, g0t)
    x0 = _cholinv(c0)
    w1t = _pass_transpose(adj, y0, x0)
    z1t = _cholapply(_gram_t(w1t), w1t)
    y1, c1 = _pass_fwd_gram(adj, z1t)
    x2 = _cholinv(c1)
    w2t = _pass_transpose(adj, y1, x2)
    z2t = _cholapply(_gram_t(w2t), w2t)
    y2, c2 = _pass_fwd_gram(adj, z2t)
    x4 = _cholinv(c2)

    q, b, zu1, zi1t, reg = _pass_six(adj, y2, x4, E_u_0, ei0t)
    e_u, e_it = _pass_seven(adj, zu1, zi1t, E_u_0, ei0t)
    t1, t2t = _t12(b, ei0t, zi1t, q, E_u_0, zu1)
    ue_u = _gu(q, t1, E_u_0, e_u)
    ue_i = _gi(b, t2t, ei0t, e_it).T

    ru, rc, rp, rn = _sc_gather_all(ue_u, ue_i, row_ids, col_ids, pos, neg)

    nlu = _log_partition(ru, e_u, 2000, table_t=False)
    nli = _log_partition(rc, e_it, 2500, table_t=True)
    loss, oth = _final(nlu, nli, reg, ru, rc, rp, rn)
    return loss[0, 0], oth[0]


# speedup vs baseline: 2.7138x; 2.7138x over previous
import functools

import jax
import jax.numpy as jnp
from jax import lax
from jax.experimental import pallas as pl
from jax.experimental.pallas import tpu as pltpu
from jax.experimental.pallas import tpu_sc as plsc

N_U = 10000
N_I = 5000
DIM = 64
TEMP = 0.2
LAMBDA_1 = 0.2
LAMBDA_2 = 1e-07
SVD_Q = 32
BATCH = 4096

BM = 1000
GRID_U = N_U // BM
F32 = jnp.float32
_HI = jax.lax.Precision.HIGHEST


def _dot(a, b, precision=None):
    return jax.lax.dot_general(a, b, (((1,), (0,)), ((), ())),
                               precision=precision, preferred_element_type=F32)


def _dot_t0(a, b):
    return jax.lax.dot_general(a, b, (((0,), (0,)), ((), ())),
                               preferred_element_type=F32)


def _dot_t1(a, b):
    return jax.lax.dot_general(a, b, (((1,), (1,)), ((), ())),
                               preferred_element_type=F32)


def _p1_body(a_ref, gt_ref, y_ref, c_ref):
    y = _dot_t1(a_ref[...], gt_ref[...])
    y_ref[...] = y

    @pl.when(pl.program_id(0) == 0)
    def _():
        c_ref[...] = jnp.zeros_like(c_ref)
    c_ref[...] += _dot_t0(y, y)


def _pass_fwd_gram(adj, gt):
    kq = gt.shape[0]
    return pl.pallas_call(
        _p1_body,
        grid=(GRID_U,),
        in_specs=[pl.BlockSpec((BM, N_I), lambda i: (i, 0)),
                  pl.BlockSpec((kq, N_I), lambda i: (0, 0))],
        out_specs=[pl.BlockSpec((BM, kq), lambda i: (i, 0)),
                   pl.BlockSpec((kq, kq), lambda i: (0, 0))],
        out_shape=[jax.ShapeDtypeStruct((N_U, kq), F32),
                   jax.ShapeDtypeStruct((kq, kq), F32)],
    )(adj, gt)


def _p2_body(a_ref, w_ref, x_ref, o_ref):
    q = _dot_t1(w_ref[...], x_ref[...])

    @pl.when(pl.program_id(0) == 0)
    def _():
        o_ref[...] = jnp.zeros_like(o_ref)
    o_ref[...] += _dot_t0(q, a_ref[...])


def _pass_transpose(adj, w, x):
    kq = w.shape[1]
    return pl.pallas_call(
        _p2_body,
        grid=(GRID_U,),
        in_specs=[pl.BlockSpec((BM, N_I), lambda i: (i, 0)),
                  pl.BlockSpec((BM, kq), lambda i: (i, 0)),
                  pl.BlockSpec((kq, kq), lambda i: (0, 0))],
        out_specs=pl.BlockSpec((kq, N_I), lambda i: (0, 0)),
        out_shape=jax.ShapeDtypeStruct((kq, N_I), F32),
    )(adj, w, x)


def _gram_t_body(wt_ref, c_ref):
    wt = wt_ref[...]
    c_ref[...] = _dot_t1(wt, wt)


def _gram_t(wt):
    kq = wt.shape[0]
    return pl.pallas_call(
        _gram_t_body,
        out_shape=jax.ShapeDtypeStruct((kq, kq), F32),
    )(wt)


def _cholinv_math(C):
    q = SVD_Q
    ri = jax.lax.broadcasted_iota(jnp.int32, (q, q), 0)
    ci = jax.lax.broadcasted_iota(jnp.int32, (q, q), 1)
    ri1 = jax.lax.broadcasted_iota(jnp.int32, (q, 1), 0)
    ci1 = jax.lax.broadcasted_iota(jnp.int32, (1, q), 1)

    def chol_step(j, carry):
        L, Ck = carry
        dj = jnp.sum(jnp.where((ri == j) & (ci == j), Ck, 0.0))
        inv_s = jax.lax.rsqrt(dj)
        col = jnp.sum(jnp.where(ci == j, Ck, 0.0), axis=1, keepdims=True)
        col = jnp.where(ri1 >= j, col, 0.0) * inv_s
        row = jnp.sum(jnp.where(ri == j, Ck, 0.0), axis=0, keepdims=True)
        row = jnp.where(ci1 >= j, row, 0.0) * inv_s
        L = L + jnp.where(ci == j, col, 0.0)
        Ck = Ck - col * row
        return L, Ck

    L, _ = jax.lax.fori_loop(0, q, chol_step, (jnp.zeros_like(C), C))

    def inv_step(i, X):
        lrow = jnp.sum(jnp.where(ri == i, L, 0.0), axis=0, keepdims=True)
        dii = jnp.sum(jnp.where(ci1 == i, lrow, 0.0))
        lrow = jnp.where(ci1 < i, lrow, 0.0)
        prod = _dot(lrow, X, precision=_HI)
        xrow = (jnp.where(ci1 == i, 1.0, 0.0) - prod) / dii
        return X + jnp.where(ri == i, xrow, 0.0)

    return jax.lax.fori_loop(0, q, inv_step, jnp.zeros_like(C))


def _cholinv_body(c_ref, x_ref):
    x_ref[...] = _cholinv_math(c_ref[...])


def _cholinv(c):
    kq = c.shape[0]
    return pl.pallas_call(
        _cholinv_body,
        out_shape=jax.ShapeDtypeStruct((kq, kq), F32),
    )(c)


def _cholapply_body(c_ref, wt_ref, zt_ref):
    zt_ref[...] = _dot(_cholinv_math(c_ref[...]), wt_ref[...])


def _cholapply(c, wt):
    kq, n = wt.shape
    return pl.pallas_call(
        _cholapply_body,
        out_shape=jax.ShapeDtypeStruct((kq, n), F32),
    )(c, wt)


def _p6_body(a_ref, w_ref, x_ref, eu0_ref, ei0t_ref,
             q_ref, b_ref, zu1_ref, zi1t_ref, reg_ref):
    qb = _dot_t1(w_ref[...], x_ref[...])
    q_ref[...] = qb
    zu1_ref[...] = _dot_t1(a_ref[...], ei0t_ref[...])

    @pl.when(pl.program_id(0) == 0)
    def _():
        b_ref[...] = jnp.zeros_like(b_ref)
        zi1t_ref[...] = jnp.zeros_like(zi1t_ref)
        reg_ref[...] = jnp.reshape(
            jnp.sum(ei0t_ref[...] * ei0t_ref[...]), (1, 1))

    b_ref[...] += _dot_t0(qb, a_ref[...])
    zi1t_ref[...] += _dot_t0(eu0_ref[...], a_ref[...])
    reg_ref[...] += jnp.reshape(jnp.sum(eu0_ref[...] * eu0_ref[...]), (1, 1))


def _pass_six(adj, w, x, eu0, ei0t):
    return pl.pallas_call(
        _p6_body,
        grid=(GRID_U,),
        in_specs=[pl.BlockSpec((BM, N_I), lambda i: (i, 0)),
                  pl.BlockSpec((BM, SVD_Q), lambda i: (i, 0)),
                  pl.BlockSpec((SVD_Q, SVD_Q), lambda i: (0, 0)),
                  pl.BlockSpec((BM, DIM), lambda i: (i, 0)),
                  pl.BlockSpec((DIM, N_I), lambda i: (0, 0))],
        out_specs=[pl.BlockSpec((BM, SVD_Q), lambda i: (i, 0)),
                   pl.BlockSpec((SVD_Q, N_I), lambda i: (0, 0)),
                   pl.BlockSpec((BM, DIM), lambda i: (i, 0)),
                   pl.BlockSpec((DIM, N_I), lambda i: (0, 0)),
                   pl.BlockSpec((1, 1), lambda i: (0, 0))],
        out_shape=[jax.ShapeDtypeStruct((N_U, SVD_Q), F32),
                   jax.ShapeDtypeStruct((SVD_Q, N_I), F32),
                   jax.ShapeDtypeStruct((N_U, DIM), F32),
                   jax.ShapeDtypeStruct((DIM, N_I), F32),
                   jax.ShapeDtypeStruct((1, 1), F32)],
    )(adj, w, x, eu0, ei0t)


def _p7_body(a_ref, zu1_ref, zi1t_ref, eu0_ref, ei0t_ref, eu_ref, eit_ref):
    eu_ref[...] = (eu0_ref[...] + zu1_ref[...]
                   + _dot_t1(a_ref[...], zi1t_ref[...]))

    @pl.when(pl.program_id(0) == 0)
    def _():
        eit_ref[...] = ei0t_ref[...] + zi1t_ref[...]
    eit_ref[...] += _dot_t0(zu1_ref[...], a_ref[...])


def _pass_seven(adj, zu1, zi1t, eu0, ei0t):
    return pl.pallas_call(
        _p7_body,
        grid=(GRID_U,),
        in_specs=[pl.BlockSpec((BM, N_I), lambda i: (i, 0)),
                  pl.BlockSpec((BM, DIM), lambda i: (i, 0)),
                  pl.BlockSpec((DIM, N_I), lambda i: (0, 0)),
                  pl.BlockSpec((BM, DIM), lambda i: (i, 0)),
                  pl.BlockSpec((DIM, N_I), lambda i: (0, 0))],
        out_specs=[pl.BlockSpec((BM, DIM), lambda i: (i, 0)),
                   pl.BlockSpec((DIM, N_I), lambda i: (0, 0))],
        out_shape=[jax.ShapeDtypeStruct((N_U, DIM), F32),
                   jax.ShapeDtypeStruct((DIM, N_I), F32)],
    )(adj, zu1, zi1t, eu0, ei0t)


def _t12_body(b_ref, ei0t_ref, zi1t_ref, q_ref, eu0_ref, zu1_ref,
              t1_ref, t2t_ref):
    @pl.when(pl.program_id(0) == 0)
    def _():
        t1_ref[...] = _dot_t1(b_ref[...], ei0t_ref[...] + zi1t_ref[...])
        t2t_ref[...] = jnp.zeros_like(t2t_ref)
    t2t_ref[...] += _dot_t0(eu0_ref[...] + zu1_ref[...], q_ref[...])


def _t12(b, ei0t, zi1t, q, eu0, zu1):
    return pl.pallas_call(
        _t12_body,
        grid=(GRID_U,),
        in_specs=[pl.BlockSpec((SVD_Q, N_I), lambda i: (0, 0)),
                  pl.BlockSpec((DIM, N_I), lambda i: (0, 0)),
                  pl.BlockSpec((DIM, N_I), lambda i: (0, 0)),
                  pl.BlockSpec((BM, SVD_Q), lambda i: (i, 0)),
                  pl.BlockSpec((BM, DIM), lambda i: (i, 0)),
                  pl.BlockSpec((BM, DIM), lambda i: (i, 0))],
        out_specs=[pl.BlockSpec((SVD_Q, DIM), lambda i: (0, 0)),
                   pl.BlockSpec((DIM, SVD_Q), lambda i: (0, 0))],
        out_shape=[jax.ShapeDtypeStruct((SVD_Q, DIM), F32),
                   jax.ShapeDtypeStruct((DIM, SVD_Q), F32)],
    )(b, ei0t, zi1t, q, eu0, zu1)


def _gu_body(q_ref, t1_ref, eu0_ref, eu_ref, out_ref):
    gu = eu0_ref[...] + _dot(q_ref[...], t1_ref[...])
    out_ref[...] = jnp.concatenate([gu, eu_ref[...]], axis=1)


def _gu(q, t1, eu0, eu):
    return pl.pallas_call(
        _gu_body,
        grid=(GRID_U,),
        in_specs=[pl.BlockSpec((BM, SVD_Q), lambda i: (i, 0)),
                  pl.BlockSpec((SVD_Q, DIM), lambda i: (0, 0)),
                  pl.BlockSpec((BM, DIM), lambda i: (i, 0)),
                  pl.BlockSpec((BM, DIM), lambda i: (i, 0))],
        out_specs=pl.BlockSpec((BM, 2 * DIM), lambda i: (i, 0)),
        out_shape=jax.ShapeDtypeStruct((N_U, 2 * DIM), F32),
    )(q, t1, eu0, eu)


def _gi_body(b_ref, t2t_ref, ei0t_ref, eit_ref, out_ref):
    git = ei0t_ref[...] + _dot(t2t_ref[...], b_ref[...])
    out_ref[...] = jnp.concatenate([git, eit_ref[...]], axis=0)


def _gi(b, t2t, ei0t, eit):
    return pl.pallas_call(
        _gi_body,
        out_shape=jax.ShapeDtypeStruct((2 * DIM, N_I), F32),
    )(b, t2t, ei0t, eit)


def _sc_gather_all(ue_u, ue_i, row_ids, col_ids, pos, neg):
    info = plsc.get_sparse_core_info()
    nc, ns = info.num_cores, info.num_subcores
    nw = nc * ns
    bpw = BATCH // nw
    mesh = plsc.VectorSubcoreMesh(core_axis_name="c", subcore_axis_name="s")
    out = jax.ShapeDtypeStruct((BATCH, 2 * DIM), F32)

    @functools.partial(
        pl.kernel, mesh=mesh,
        out_type=(out,) * 4,
        scratch_types=[pltpu.VMEM((bpw,), jnp.int32),
                       pltpu.VMEM((bpw, 2 * DIM), F32),
                       pltpu.SemaphoreType.DMA],
    )
    def k(tu_hbm, ti_hbm, rid_hbm, cid_hbm, pos_hbm, neg_hbm,
          o_ru, o_rc, o_rp, o_rn, idx_v, rows_v, sem):
        wid = lax.axis_index("s") * nc + lax.axis_index("c")
        base = wid * bpw

        def gather(idx_hbm, table_hbm, out_hbm):
            pltpu.sync_copy(idx_hbm.at[pl.ds(base, bpw)], idx_v)
            pltpu.async_copy(table_hbm.at[idx_v], rows_v, sem).wait()
            pltpu.sync_copy(rows_v, out_hbm.at[pl.ds(base, bpw)])

        gather(rid_hbm, tu_hbm, o_ru)
        gather(cid_hbm, ti_hbm, o_rc)
        gather(pos_hbm, ti_hbm, o_rp)
        gather(neg_hbm, ti_hbm, o_rn)

    return k(ue_u, ue_i, row_ids, col_ids, pos, neg)


def _lse_body(n_chunks, chunk, table_t, g_ref, e_ref, o_ref):
    g = g_ref[:, :DIM]
    s = jnp.zeros((g.shape[0], 1), F32)
    for kc in range(n_chunks):
        if table_t:
            logits = _dot(g, e_ref[:, kc * chunk:(kc + 1) * chunk])
        else:
            logits = _dot_t1(g, e_ref[kc * chunk:(kc + 1) * chunk, :])
        s = s + jnp.sum(jnp.exp(logits * (1.0 / TEMP)), axis=1, keepdims=True)

    @pl.when(pl.program_id(0) == 0)
    def _():
        o_ref[...] = jnp.zeros_like(o_ref)
    o_ref[...] += jnp.reshape(jnp.sum(jnp.log(s + 1e-08)), (1, 1))


def _log_partition(rows, table, chunk, table_t):
    n = table.shape[1] if table_t else table.shape[0]
    bb = 512
    body = functools.partial(_lse_body, n // chunk, chunk, table_t)
    return pl.pallas_call(
        body,
        grid=(BATCH // bb,),
        in_specs=[pl.BlockSpec((bb, 2 * DIM), lambda i: (i, 0)),
                  pl.BlockSpec(table.shape, lambda i: (0, 0))],
        out_specs=pl.BlockSpec((1, 1), lambda i: (0, 0)),
        out_shape=jax.ShapeDtypeStruct((1, 1), F32),
    )(rows, table)


def _final_body(nlu_ref, nli_ref, reg_ref, ru_ref, rc_ref, rp_ref, rn_ref,
                loss_ref, oth_ref):
    inv_b = 1.0 / BATCH
    gur, eur = ru_ref[:, :DIM], ru_ref[:, DIM:]
    gic, eic = rc_ref[:, :DIM], rc_ref[:, DIM:]
    eip, ein = rp_ref[:, DIM:], rn_ref[:, DIM:]

    neg_score = (nlu_ref[0, 0] + nli_ref[0, 0]) * inv_b
    pu = jnp.sum(gur * eur, axis=1, keepdims=True) * (1.0 / TEMP)
    pi = jnp.sum(gic * eic, axis=1, keepdims=True) * (1.0 / TEMP)
    pos_score = (jnp.sum(jnp.clip(pu, -5.0, 5.0)) +
                 jnp.sum(jnp.clip(pi, -5.0, 5.0))) * inv_b
    loss_cl = -pos_score + neg_score

    d = jnp.sum(eur * eip, axis=1, keepdims=True) - \
        jnp.sum(eur * ein, axis=1, keepdims=True)
    loss_bpr = jnp.sum(jnp.log(1.0 + jnp.exp(-d))) * inv_b

    loss = loss_bpr + LAMBDA_1 * loss_cl + LAMBDA_2 * reg_ref[0, 0]
    loss_ref[...] = jnp.reshape(loss, (1, 1))
    oth_ref[...] = jnp.concatenate(
        [jnp.full((1, 1), loss_bpr, F32),
         jnp.full((1, 1), LAMBDA_1 * loss_cl, F32)], axis=1)


def _final(nlu, nli, reg, ru, rc, rp, rn):
    bspec = pl.BlockSpec((BATCH, 2 * DIM), lambda: (0, 0))
    sspec = pl.BlockSpec((1, 1), lambda: (0, 0))
    return pl.pallas_call(
        _final_body,
        in_specs=[sspec, sspec, sspec] + [bspec] * 4,
        out_specs=[sspec, pl.BlockSpec((1, 2), lambda: (0, 0))],
        out_shape=[jax.ShapeDtypeStruct((1, 1), F32),
                   jax.ShapeDtypeStruct((1, 2), F32)],
    )(nlu, nli, reg, ru, rc, rp, rn)


def kernel(adj, row_ids, col_ids, pos, neg, E_u_0, E_i_0):
    g0t = jax.random.normal(jax.random.key(42), (N_I, SVD_Q), dtype=F32).T
    ei0t = E_i_0.T

    y0, c0 = _pass_fwd_gram(adj, g0t)
    x0 = _cholinv(c0)
    w1t = _pass_transpose(adj, y0, x0)
    z1t = _cholapply(_gram_t(w1t), w1t)
    y1, c1 = _pass_fwd_gram(adj, z1t)
    x2 = _cholinv(c1)
    w2t = _pass_transpose(adj, y1, x2)
    z2t = _cholapply(_gram_t(w2t), w2t)
    y2, c2 = _pass_fwd_gram(adj, z2t)
    x4 = _cholinv(c2)

    q, b, zu1, zi1t, reg = _pass_six(adj, y2, x4, E_u_0, ei0t)
    e_u, e_it = _pass_seven(adj, zu1, zi1t, E_u_0, ei0t)
    t1, t2t = _t12(b, ei0t, zi1t, q, E_u_0, zu1)
    ue_u = _gu(q, t1, E_u_0, e_u)
    ue_i = _gi(b, t2t, ei0t, e_it).T

    ru, rc, rp, rn = _sc_gather_all(ue_u, ue_i, row_ids, col_ids, pos, neg)

    nlu = _log_partition(ru, e_u, 2000, table_t=False)
    nli = _log_partition(rc, e_it, 2500, table_t=True)
    loss, oth = _final(nlu, nli, reg, ru, rc, rp, rn)
    return loss[0, 0], oth[0]


# speedup vs baseline: 3.1074x; 1.1450x over previous
import functools

import jax
import jax.numpy as jnp
from jax import lax
from jax.experimental import pallas as pl
from jax.experimental.pallas import tpu as pltpu
from jax.experimental.pallas import tpu_sc as plsc

N_U = 10000
N_I = 5000
DIM = 64
TEMP = 0.2
LAMBDA_1 = 0.2
LAMBDA_2 = 1e-07
SVD_Q = 32
BATCH = 4096

BM = 1000
GRID_U = N_U // BM
F32 = jnp.float32
_HI = jax.lax.Precision.HIGHEST


def _dot(a, b, precision=None):
    return jax.lax.dot_general(a, b, (((1,), (0,)), ((), ())),
                               precision=precision, preferred_element_type=F32)


def _dot_t0(a, b):
    return jax.lax.dot_general(a, b, (((0,), (0,)), ((), ())),
                               preferred_element_type=F32)


def _dot_t1(a, b):
    return jax.lax.dot_general(a, b, (((1,), (1,)), ((), ())),
                               preferred_element_type=F32)


def _p1_body(a_ref, gt_ref, y_ref, c_ref):
    y = _dot_t1(a_ref[...], gt_ref[...])
    y_ref[...] = y

    @pl.when(pl.program_id(0) == 0)
    def _():
        c_ref[...] = jnp.zeros_like(c_ref)
    c_ref[...] += _dot_t0(y, y)


def _pass_fwd_gram(adj, gt):
    kq = gt.shape[0]
    return pl.pallas_call(
        _p1_body,
        grid=(GRID_U,),
        in_specs=[pl.BlockSpec((BM, N_I), lambda i: (i, 0)),
                  pl.BlockSpec((kq, N_I), lambda i: (0, 0))],
        out_specs=[pl.BlockSpec((BM, kq), lambda i: (i, 0)),
                   pl.BlockSpec((kq, kq), lambda i: (0, 0))],
        out_shape=[jax.ShapeDtypeStruct((N_U, kq), F32),
                   jax.ShapeDtypeStruct((kq, kq), F32)],
    )(adj, gt)


def _p2_body(a_ref, w_ref, x_ref, o_ref):
    q = _dot_t1(w_ref[...], x_ref[...])

    @pl.when(pl.program_id(0) == 0)
    def _():
        o_ref[...] = jnp.zeros_like(o_ref)
    o_ref[...] += _dot_t0(q, a_ref[...])


def _pass_transpose(adj, w, x):
    kq = w.shape[1]
    return pl.pallas_call(
        _p2_body,
        grid=(GRID_U,),
        in_specs=[pl.BlockSpec((BM, N_I), lambda i: (i, 0)),
                  pl.BlockSpec((BM, kq), lambda i: (i, 0)),
                  pl.BlockSpec((kq, kq), lambda i: (0, 0))],
        out_specs=pl.BlockSpec((kq, N_I), lambda i: (0, 0)),
        out_shape=jax.ShapeDtypeStruct((kq, N_I), F32),
    )(adj, w, x)


def _cholinv_math(C):
    q = SVD_Q
    ri = jax.lax.broadcasted_iota(jnp.int32, (q, q), 0)
    ci = jax.lax.broadcasted_iota(jnp.int32, (q, q), 1)
    ri1 = jax.lax.broadcasted_iota(jnp.int32, (q, 1), 0)
    ci1 = jax.lax.broadcasted_iota(jnp.int32, (1, q), 1)

    def chol_step(j, carry):
        L, Ck = carry
        dj = jnp.sum(jnp.where((ri == j) & (ci == j), Ck, 0.0))
        inv_s = jax.lax.rsqrt(dj)
        col = jnp.sum(jnp.where(ci == j, Ck, 0.0), axis=1, keepdims=True)
        col = jnp.where(ri1 >= j, col, 0.0) * inv_s
        row = jnp.sum(jnp.where(ri == j, Ck, 0.0), axis=0, keepdims=True)
        row = jnp.where(ci1 >= j, row, 0.0) * inv_s
        L = L + jnp.where(ci == j, col, 0.0)
        Ck = Ck - col * row
        return L, Ck

    L, _ = jax.lax.fori_loop(0, q, chol_step, (jnp.zeros_like(C), C))

    def inv_step(i, X):
        lrow = jnp.sum(jnp.where(ri == i, L, 0.0), axis=0, keepdims=True)
        dii = jnp.sum(jnp.where(ci1 == i, lrow, 0.0))
        lrow = jnp.where(ci1 < i, lrow, 0.0)
        prod = _dot(lrow, X, precision=_HI)
        xrow = (jnp.where(ci1 == i, 1.0, 0.0) - prod) / dii
        return X + jnp.where(ri == i, xrow, 0.0)

    return jax.lax.fori_loop(0, q, inv_step, jnp.zeros_like(C))


def _cholinv_body(c_ref, x_ref):
    x_ref[...] = _cholinv_math(c_ref[...])


def _cholinv(c):
    kq = c.shape[0]
    return pl.pallas_call(
        _cholinv_body,
        out_shape=jax.ShapeDtypeStruct((kq, kq), F32),
    )(c)


def _chol_orth_body(wt_ref, zt_ref):
    wt = wt_ref[...]
    x = _cholinv_math(_dot_t1(wt, wt))
    zt_ref[...] = _dot(x, wt)


def _chol_orth(wt):
    kq, n = wt.shape
    return pl.pallas_call(
        _chol_orth_body,
        out_shape=jax.ShapeDtypeStruct((kq, n), F32),
    )(wt)


def _p5_body(a_ref, zt_ref, eu0_ref, ei0t_ref,
             y_ref, c_ref, bt_ref, zu1_ref, zi1t_ref, reg_ref):
    y = _dot_t1(a_ref[...], zt_ref[...])
    y_ref[...] = y
    zu1_ref[...] = _dot_t1(a_ref[...], ei0t_ref[...])

    @pl.when(pl.program_id(0) == 0)
    def _():
        c_ref[...] = jnp.zeros_like(c_ref)
        bt_ref[...] = jnp.zeros_like(bt_ref)
        zi1t_ref[...] = jnp.zeros_like(zi1t_ref)
        reg_ref[...] = jnp.reshape(
            jnp.sum(ei0t_ref[...] * ei0t_ref[...]), (1, 1))

    c_ref[...] += _dot_t0(y, y)
    bt_ref[...] += _dot_t0(y, a_ref[...])
    zi1t_ref[...] += _dot_t0(eu0_ref[...], a_ref[...])
    reg_ref[...] += jnp.reshape(jnp.sum(eu0_ref[...] * eu0_ref[...]), (1, 1))


def _pass_five(adj, zt, eu0, ei0t):
    return pl.pallas_call(
        _p5_body,
        grid=(GRID_U,),
        in_specs=[pl.BlockSpec((BM, N_I), lambda i: (i, 0)),
                  pl.BlockSpec((SVD_Q, N_I), lambda i: (0, 0)),
                  pl.BlockSpec((BM, DIM), lambda i: (i, 0)),
                  pl.BlockSpec((DIM, N_I), lambda i: (0, 0))],
        out_specs=[pl.BlockSpec((BM, SVD_Q), lambda i: (i, 0)),
                   pl.BlockSpec((SVD_Q, SVD_Q), lambda i: (0, 0)),
                   pl.BlockSpec((SVD_Q, N_I), lambda i: (0, 0)),
                   pl.BlockSpec((BM, DIM), lambda i: (i, 0)),
                   pl.BlockSpec((DIM, N_I), lambda i: (0, 0)),
                   pl.BlockSpec((1, 1), lambda i: (0, 0))],
        out_shape=[jax.ShapeDtypeStruct((N_U, SVD_Q), F32),
                   jax.ShapeDtypeStruct((SVD_Q, SVD_Q), F32),
                   jax.ShapeDtypeStruct((SVD_Q, N_I), F32),
                   jax.ShapeDtypeStruct((N_U, DIM), F32),
                   jax.ShapeDtypeStruct((DIM, N_I), F32),
                   jax.ShapeDtypeStruct((1, 1), F32)],
    )(adj, zt, eu0, ei0t)


def _cholmt_body(c_ref, bt_ref, ei0t_ref, zi1t_ref, m_ref, t1_ref):
    x = _cholinv_math(c_ref[...])
    m = _dot_t0(x, x)
    m_ref[...] = m
    t1_ref[...] = _dot(m, _dot_t1(bt_ref[...], ei0t_ref[...] + zi1t_ref[...]))


def _cholmt(c, bt, ei0t, zi1t):
    return pl.pallas_call(
        _cholmt_body,
        out_shape=[jax.ShapeDtypeStruct((SVD_Q, SVD_Q), F32),
                   jax.ShapeDtypeStruct((SVD_Q, DIM), F32)],
    )(c, bt, ei0t, zi1t)


def _p7_body(a_ref, y_ref, t1_ref, zu1_ref, zi1t_ref, eu0_ref, ei0t_ref,
             ueu_ref, eit_ref, t2t_ref):
    zu1 = zu1_ref[...]
    eu0 = eu0_ref[...]
    e_u = eu0 + zu1 + _dot_t1(a_ref[...], zi1t_ref[...])
    g_u = eu0 + _dot(y_ref[...], t1_ref[...])
    ueu_ref[...] = jnp.concatenate([g_u, e_u], axis=1)

    @pl.when(pl.program_id(0) == 0)
    def _():
        eit_ref[...] = ei0t_ref[...] + zi1t_ref[...]
        t2t_ref[...] = jnp.zeros_like(t2t_ref)
    eit_ref[...] += _dot_t0(zu1, a_ref[...])
    t2t_ref[...] += _dot_t0(eu0 + zu1, y_ref[...])


def _pass_seven(adj, y2, t1, zu1, zi1t, eu0, ei0t):
    return pl.pallas_call(
        _p7_body,
        grid=(GRID_U,),
        in_specs=[pl.BlockSpec((BM, N_I), lambda i: (i, 0)),
                  pl.BlockSpec((BM, SVD_Q), lambda i: (i, 0)),
                  pl.BlockSpec((SVD_Q, DIM), lambda i: (0, 0)),
                  pl.BlockSpec((BM, DIM), lambda i: (i, 0)),
                  pl.BlockSpec((DIM, N_I), lambda i: (0, 0)),
                  pl.BlockSpec((BM, DIM), lambda i: (i, 0)),
                  pl.BlockSpec((DIM, N_I), lambda i: (0, 0))],
        out_specs=[pl.BlockSpec((BM, 2 * DIM), lambda i: (i, 0)),
                   pl.BlockSpec((DIM, N_I), lambda i: (0, 0)),
                   pl.BlockSpec((DIM, SVD_Q), lambda i: (0, 0))],
        out_shape=[jax.ShapeDtypeStruct((N_U, 2 * DIM), F32),
                   jax.ShapeDtypeStruct((DIM, N_I), F32),
                   jax.ShapeDtypeStruct((DIM, SVD_Q), F32)],
    )(adj, y2, t1, zu1, zi1t, eu0, ei0t)


def _gi_body(bt_ref, m_ref, t2t_ref, ei0t_ref, eit_ref, out_ref):
    git = ei0t_ref[...] + _dot(_dot(t2t_ref[...], m_ref[...]), bt_ref[...])
    out_ref[...] = jnp.concatenate([git, eit_ref[...]], axis=0)


def _gi(bt, m, t2t, ei0t, eit):
    return pl.pallas_call(
        _gi_body,
        out_shape=jax.ShapeDtypeStruct((2 * DIM, N_I), F32),
    )(bt, m, t2t, ei0t, eit)


def _sc_gather_all(ue_u, ue_i, row_ids, col_ids, pos, neg):
    info = plsc.get_sparse_core_info()
    nc, ns = info.num_cores, info.num_subcores
    nw = nc * ns
    bpw = BATCH // nw
    mesh = plsc.VectorSubcoreMesh(core_axis_name="c", subcore_axis_name="s")
    out = jax.ShapeDtypeStruct((BATCH, 2 * DIM), F32)

    @functools.partial(
        pl.kernel, mesh=mesh,
        out_type=(out,) * 4,
        scratch_types=[pltpu.VMEM((bpw,), jnp.int32),
                       pltpu.VMEM((bpw, 2 * DIM), F32),
                       pltpu.SemaphoreType.DMA],
    )
    def k(tu_hbm, ti_hbm, rid_hbm, cid_hbm, pos_hbm, neg_hbm,
          o_ru, o_rc, o_rp, o_rn, idx_v, rows_v, sem):
        wid = lax.axis_index("s") * nc + lax.axis_index("c")
        base = wid * bpw

        def gather(idx_hbm, table_hbm, out_hbm):
            pltpu.sync_copy(idx_hbm.at[pl.ds(base, bpw)], idx_v)
            pltpu.async_copy(table_hbm.at[idx_v], rows_v, sem).wait()
            pltpu.sync_copy(rows_v, out_hbm.at[pl.ds(base, bpw)])

        gather(rid_hbm, tu_hbm, o_ru)
        gather(cid_hbm, ti_hbm, o_rc)
        gather(pos_hbm, ti_hbm, o_rp)
        gather(neg_hbm, ti_hbm, o_rn)

    return k(ue_u, ue_i, row_ids, col_ids, pos, neg)


_BB = 512
_NB = BATCH // _BB
_UCH = 2000
_ICH = 2500


def _loss_body(ueu_ref, eit_ref, reg_ref, ru_ref, rc_ref, rp_ref, rn_ref,
               loss_ref, oth_ref, acc_ref):
    i = pl.program_id(0)
    gur, eur = ru_ref[:, :DIM], ru_ref[:, DIM:]
    gic, eic = rc_ref[:, :DIM], rc_ref[:, DIM:]
    eip, ein = rp_ref[:, DIM:], rn_ref[:, DIM:]

    su = jnp.zeros((_BB, 1), F32)
    for kc in range(N_U // _UCH):
        logits = _dot_t1(gur, ueu_ref[kc * _UCH:(kc + 1) * _UCH, DIM:])
        su = su + jnp.sum(jnp.exp(logits * (1.0 / TEMP)), axis=1, keepdims=True)
    si = jnp.zeros((_BB, 1), F32)
    for kc in range(N_I // _ICH):
        logits = _dot(gic, eit_ref[:, kc * _ICH:(kc + 1) * _ICH])
        si = si + jnp.sum(jnp.exp(logits * (1.0 / TEMP)), axis=1, keepdims=True)
    nl_u = jnp.sum(jnp.log(su + 1e-08))
    nl_i = jnp.sum(jnp.log(si + 1e-08))

    pu = jnp.sum(gur * eur, axis=1, keepdims=True) * (1.0 / TEMP)
    pi = jnp.sum(gic * eic, axis=1, keepdims=True) * (1.0 / TEMP)
    pos = jnp.sum(jnp.clip(pu, -5.0, 5.0)) + jnp.sum(jnp.clip(pi, -5.0, 5.0))
    d = jnp.sum(eur * eip, axis=1, keepdims=True) - \
        jnp.sum(eur * ein, axis=1, keepdims=True)
    bpr = jnp.sum(jnp.log(1.0 + jnp.exp(-d)))

    upd = jnp.concatenate(
        [jnp.reshape(nl_u, (1, 1)), jnp.reshape(nl_i, (1, 1)),
         jnp.reshape(pos, (1, 1)), jnp.reshape(bpr, (1, 1)),
         jnp.zeros((1, 124), F32)], axis=1)

    @pl.when(i == 0)
    def _():
        acc_ref[...] = jnp.zeros_like(acc_ref)
    acc_ref[...] += upd

    @pl.when(i == _NB - 1)
    def _():
        acc = acc_ref[...]
        inv_b = 1.0 / BATCH
        neg_score = (acc[0, 0] + acc[0, 1]) * inv_b
        pos_score = acc[0, 2] * inv_b
        loss_bpr = acc[0, 3] * inv_b
        loss_cl = -pos_score + neg_score
        loss = loss_bpr + LAMBDA_1 * loss_cl + LAMBDA_2 * reg_ref[0, 0]
        loss_ref[...] = jnp.reshape(loss, (1, 1))
        oth_ref[...] = jnp.concatenate(
            [jnp.full((1, 1), loss_bpr, F32),
             jnp.full((1, 1), LAMBDA_1 * loss_cl, F32)], axis=1)


def _loss(ueu, eit, reg, ru, rc, rp, rn):
    bspec = pl.BlockSpec((_BB, 2 * DIM), lambda i: (i, 0))
    return pl.pallas_call(
        _loss_body,
        grid=(_NB,),
        in_specs=[pl.BlockSpec((N_U, 2 * DIM), lambda i: (0, 0)),
                  pl.BlockSpec((DIM, N_I), lambda i: (0, 0)),
                  pl.BlockSpec((1, 1), lambda i: (0, 0)),
                  bspec, bspec, bspec, bspec],
        out_specs=[pl.BlockSpec((1, 1), lambda i: (0, 0)),
                   pl.BlockSpec((1, 2), lambda i: (0, 0))],
        out_shape=[jax.ShapeDtypeStruct((1, 1), F32),
                   jax.ShapeDtypeStruct((1, 2), F32)],
        scratch_shapes=[pltpu.VMEM((1, 128), F32)],
    )(ueu, eit, reg, ru, rc, rp, rn)


def kernel(adj, row_ids, col_ids, pos, neg, E_u_0, E_i_0):
    g0t = jax.random.normal(jax.random.key(42), (N_I, SVD_Q), dtype=F32).T
    ei0t = E_i_0.T

    y0, c0 = _pass_fwd_gram(adj, g0t)
    x0 = _cholinv(c0)
    z1t = _chol_orth(_pass_transpose(adj, y0, x0))
    y1, c1 = _pass_fwd_gram(adj, z1t)
    x2 = _cholinv(c1)
    z2t = _chol_orth(_pass_transpose(adj, y1, x2))

    y2, c4, bt, zu1, zi1t, reg = _pass_five(adj, z2t, E_u_0, ei0t)
    m, t1 = _cholmt(c4, bt, ei0t, zi1t)
    ue_u, e_it, t2t = _pass_seven(adj, y2, t1, zu1, zi1t, E_u_0, ei0t)
    ue_i = _gi(bt, m, t2t, ei0t, e_it).T

    ru, rc, rp, rn = _sc_gather_all(ue_u, ue_i, row_ids, col_ids, pos, neg)

    loss, oth = _loss(ue_u, e_it, reg, ru, rc, rp, rn)
    return loss[0, 0], oth[0]


# speedup vs baseline: 3.4166x; 1.0995x over previous
import functools

import jax
import jax.numpy as jnp
from jax import lax
from jax.experimental import pallas as pl
from jax.experimental.pallas import tpu as pltpu
from jax.experimental.pallas import tpu_sc as plsc

N_U = 10000
N_I = 5000
DIM = 64
TEMP = 0.2
LAMBDA_1 = 0.2
LAMBDA_2 = 1e-07
SVD_Q = 32
BATCH = 4096

BM = 1000
GRID_U = N_U // BM
F32 = jnp.float32
_HI = jax.lax.Precision.HIGHEST


def _dot(a, b, precision=None):
    return jax.lax.dot_general(a, b, (((1,), (0,)), ((), ())),
                               precision=precision, preferred_element_type=F32)


def _dot_t0(a, b):
    return jax.lax.dot_general(a, b, (((0,), (0,)), ((), ())),
                               preferred_element_type=F32)


def _dot_t1(a, b):
    return jax.lax.dot_general(a, b, (((1,), (1,)), ((), ())),
                               preferred_element_type=F32)


BF16 = jnp.bfloat16


def _p0_body(a_ref, gt_ref, y_ref, c_ref, a16_ref):
    a = a_ref[...]
    a16_ref[...] = a.astype(BF16)
    y = _dot_t1(a, gt_ref[...])
    y_ref[...] = y

    @pl.when(pl.program_id(0) == 0)
    def _():
        c_ref[...] = jnp.zeros_like(c_ref)
    c_ref[...] += _dot_t0(y, y)


def _pass_first(adj, gt):
    kq = gt.shape[0]
    bm0 = 400
    return pl.pallas_call(
        _p0_body,
        grid=(N_U // bm0,),
        in_specs=[pl.BlockSpec((bm0, N_I), lambda i: (i, 0)),
                  pl.BlockSpec((kq, N_I), lambda i: (0, 0))],
        out_specs=[pl.BlockSpec((bm0, kq), lambda i: (i, 0)),
                   pl.BlockSpec((kq, kq), lambda i: (0, 0)),
                   pl.BlockSpec((bm0, N_I), lambda i: (i, 0))],
        out_shape=[jax.ShapeDtypeStruct((N_U, kq), F32),
                   jax.ShapeDtypeStruct((kq, kq), F32),
                   jax.ShapeDtypeStruct((N_U, N_I), BF16)],
    )(adj, gt)


def _p1_body(a_ref, gt_ref, y_ref, c_ref):
    y = _dot_t1(a_ref[...], gt_ref[...].astype(BF16))
    y_ref[...] = y

    @pl.when(pl.program_id(0) == 0)
    def _():
        c_ref[...] = jnp.zeros_like(c_ref)
    c_ref[...] += _dot_t0(y, y)


def _pass_fwd_gram(a16, gt):
    kq = gt.shape[0]
    return pl.pallas_call(
        _p1_body,
        grid=(GRID_U,),
        in_specs=[pl.BlockSpec((BM, N_I), lambda i: (i, 0)),
                  pl.BlockSpec((kq, N_I), lambda i: (0, 0))],
        out_specs=[pl.BlockSpec((BM, kq), lambda i: (i, 0)),
                   pl.BlockSpec((kq, kq), lambda i: (0, 0))],
        out_shape=[jax.ShapeDtypeStruct((N_U, kq), F32),
                   jax.ShapeDtypeStruct((kq, kq), F32)],
    )(a16, gt)


def _p2_body(a_ref, w_ref, x_ref, o_ref):
    q = _dot_t1(w_ref[...], x_ref[...])

    @pl.when(pl.program_id(0) == 0)
    def _():
        o_ref[...] = jnp.zeros_like(o_ref)
    o_ref[...] += _dot_t0(q.astype(BF16), a_ref[...])


def _pass_transpose(adj, w, x):
    kq = w.shape[1]
    return pl.pallas_call(
        _p2_body,
        grid=(GRID_U,),
        in_specs=[pl.BlockSpec((BM, N_I), lambda i: (i, 0)),
                  pl.BlockSpec((BM, kq), lambda i: (i, 0)),
                  pl.BlockSpec((kq, kq), lambda i: (0, 0))],
        out_specs=pl.BlockSpec((kq, N_I), lambda i: (0, 0)),
        out_shape=jax.ShapeDtypeStruct((kq, N_I), F32),
    )(adj, w, x)


def _cholinv_math(C):
    q = SVD_Q
    ri = jax.lax.broadcasted_iota(jnp.int32, (q, q), 0)
    ci = jax.lax.broadcasted_iota(jnp.int32, (q, q), 1)
    ri1 = jax.lax.broadcasted_iota(jnp.int32, (q, 1), 0)
    ci1 = jax.lax.broadcasted_iota(jnp.int32, (1, q), 1)

    def chol_step(j, carry):
        L, Ck = carry
        dj = jnp.sum(jnp.where((ri == j) & (ci == j), Ck, 0.0))
        inv_s = jax.lax.rsqrt(dj)
        col = jnp.sum(jnp.where(ci == j, Ck, 0.0), axis=1, keepdims=True)
        col = jnp.where(ri1 >= j, col, 0.0) * inv_s
        row = jnp.sum(jnp.where(ri == j, Ck, 0.0), axis=0, keepdims=True)
        row = jnp.where(ci1 >= j, row, 0.0) * inv_s
        L = L + jnp.where(ci == j, col, 0.0)
        Ck = Ck - col * row
        return L, Ck

    L, _ = jax.lax.fori_loop(0, q, chol_step, (jnp.zeros_like(C), C))

    def inv_step(i, X):
        lrow = jnp.sum(jnp.where(ri == i, L, 0.0), axis=0, keepdims=True)
        dii = jnp.sum(jnp.where(ci1 == i, lrow, 0.0))
        lrow = jnp.where(ci1 < i, lrow, 0.0)
        prod = _dot(lrow, X, precision=_HI)
        xrow = (jnp.where(ci1 == i, 1.0, 0.0) - prod) / dii
        return X + jnp.where(ri == i, xrow, 0.0)

    return jax.lax.fori_loop(0, q, inv_step, jnp.zeros_like(C))


def _cholinv_body(c_ref, x_ref):
    x_ref[...] = _cholinv_math(c_ref[...])


def _cholinv(c):
    kq = c.shape[0]
    return pl.pallas_call(
        _cholinv_body,
        out_shape=jax.ShapeDtypeStruct((kq, kq), F32),
    )(c)


def _chol_orth_body(wt_ref, zt_ref):
    wt = wt_ref[...]
    x = _cholinv_math(_dot_t1(wt, wt))
    zt_ref[...] = _dot(x, wt)


def _chol_orth(wt):
    kq, n = wt.shape
    return pl.pallas_call(
        _chol_orth_body,
        out_shape=jax.ShapeDtypeStruct((kq, n), F32),
    )(wt)


def _p5_body(a_ref, zt_ref, eu0_ref, ei0t_ref,
             y_ref, c_ref, bt_ref, zu1_ref, zi1t_ref, reg_ref):
    y = _dot_t1(a_ref[...], zt_ref[...].astype(BF16))
    y_ref[...] = y
    zu1_ref[...] = _dot_t1(a_ref[...], ei0t_ref[...].astype(BF16))

    @pl.when(pl.program_id(0) == 0)
    def _():
        c_ref[...] = jnp.zeros_like(c_ref)
        bt_ref[...] = jnp.zeros_like(bt_ref)
        zi1t_ref[...] = jnp.zeros_like(zi1t_ref)
        reg_ref[...] = jnp.reshape(
            jnp.sum(ei0t_ref[...] * ei0t_ref[...]), (1, 1))

    c_ref[...] += _dot_t0(y, y)
    bt_ref[...] += _dot_t0(y.astype(BF16), a_ref[...])
    zi1t_ref[...] += _dot_t0(eu0_ref[...].astype(BF16), a_ref[...])
    reg_ref[...] += jnp.reshape(jnp.sum(eu0_ref[...] * eu0_ref[...]), (1, 1))


def _pass_five(adj, zt, eu0, ei0t):
    return pl.pallas_call(
        _p5_body,
        grid=(GRID_U,),
        in_specs=[pl.BlockSpec((BM, N_I), lambda i: (i, 0)),
                  pl.BlockSpec((SVD_Q, N_I), lambda i: (0, 0)),
                  pl.BlockSpec((BM, DIM), lambda i: (i, 0)),
                  pl.BlockSpec((DIM, N_I), lambda i: (0, 0))],
        out_specs=[pl.BlockSpec((BM, SVD_Q), lambda i: (i, 0)),
                   pl.BlockSpec((SVD_Q, SVD_Q), lambda i: (0, 0)),
                   pl.BlockSpec((SVD_Q, N_I), lambda i: (0, 0)),
                   pl.BlockSpec((BM, DIM), lambda i: (i, 0)),
                   pl.BlockSpec((DIM, N_I), lambda i: (0, 0)),
                   pl.BlockSpec((1, 1), lambda i: (0, 0))],
        out_shape=[jax.ShapeDtypeStruct((N_U, SVD_Q), F32),
                   jax.ShapeDtypeStruct((SVD_Q, SVD_Q), F32),
                   jax.ShapeDtypeStruct((SVD_Q, N_I), F32),
                   jax.ShapeDtypeStruct((N_U, DIM), F32),
                   jax.ShapeDtypeStruct((DIM, N_I), F32),
                   jax.ShapeDtypeStruct((1, 1), F32)],
    )(adj, zt, eu0, ei0t)


def _cholmt_body(c_ref, bt_ref, ei0t_ref, zi1t_ref, m_ref, t1_ref):
    x = _cholinv_math(c_ref[...])
    m = _dot_t0(x, x)
    m_ref[...] = m
    t1_ref[...] = _dot(m, _dot_t1(bt_ref[...], ei0t_ref[...] + zi1t_ref[...]))


def _cholmt(c, bt, ei0t, zi1t):
    return pl.pallas_call(
        _cholmt_body,
        out_shape=[jax.ShapeDtypeStruct((SVD_Q, SVD_Q), F32),
                   jax.ShapeDtypeStruct((SVD_Q, DIM), F32)],
    )(c, bt, ei0t, zi1t)


def _p7_body(a_ref, y_ref, t1_ref, zu1_ref, zi1t_ref, eu0_ref, ei0t_ref,
             ueu_ref, eit_ref, t2t_ref):
    zu1 = zu1_ref[...]
    eu0 = eu0_ref[...]
    e_u = eu0 + zu1 + _dot_t1(a_ref[...], zi1t_ref[...].astype(BF16))
    g_u = eu0 + _dot(y_ref[...], t1_ref[...])
    ueu_ref[...] = jnp.concatenate([g_u, e_u], axis=1)

    @pl.when(pl.program_id(0) == 0)
    def _():
        eit_ref[...] = ei0t_ref[...] + zi1t_ref[...]
        t2t_ref[...] = jnp.zeros_like(t2t_ref)
    eit_ref[...] += _dot_t0(zu1.astype(BF16), a_ref[...])
    t2t_ref[...] += _dot_t0(eu0 + zu1, y_ref[...])


def _pass_seven(adj, y2, t1, zu1, zi1t, eu0, ei0t):
    return pl.pallas_call(
        _p7_body,
        grid=(GRID_U,),
        in_specs=[pl.BlockSpec((BM, N_I), lambda i: (i, 0)),
                  pl.BlockSpec((BM, SVD_Q), lambda i: (i, 0)),
                  pl.BlockSpec((SVD_Q, DIM), lambda i: (0, 0)),
                  pl.BlockSpec((BM, DIM), lambda i: (i, 0)),
                  pl.BlockSpec((DIM, N_I), lambda i: (0, 0)),
                  pl.BlockSpec((BM, DIM), lambda i: (i, 0)),
                  pl.BlockSpec((DIM, N_I), lambda i: (0, 0))],
        out_specs=[pl.BlockSpec((BM, 2 * DIM), lambda i: (i, 0)),
                   pl.BlockSpec((DIM, N_I), lambda i: (0, 0)),
                   pl.BlockSpec((DIM, SVD_Q), lambda i: (0, 0))],
        out_shape=[jax.ShapeDtypeStruct((N_U, 2 * DIM), F32),
                   jax.ShapeDtypeStruct((DIM, N_I), F32),
                   jax.ShapeDtypeStruct((DIM, SVD_Q), F32)],
    )(adj, y2, t1, zu1, zi1t, eu0, ei0t)


def _gi_body(bt_ref, m_ref, t2t_ref, ei0t_ref, eit_ref, out_ref):
    git = ei0t_ref[...] + _dot(_dot(t2t_ref[...], m_ref[...]), bt_ref[...])
    out_ref[...] = jnp.concatenate([git, eit_ref[...]], axis=0)


def _gi(bt, m, t2t, ei0t, eit):
    return pl.pallas_call(
        _gi_body,
        out_shape=jax.ShapeDtypeStruct((2 * DIM, N_I), F32),
    )(bt, m, t2t, ei0t, eit)


def _sc_gather_all(ue_u, ue_i, row_ids, col_ids, pos, neg):
    info = plsc.get_sparse_core_info()
    nc, ns = info.num_cores, info.num_subcores
    nw = nc * ns
    bpw = BATCH // nw
    mesh = plsc.VectorSubcoreMesh(core_axis_name="c", subcore_axis_name="s")
    out = jax.ShapeDtypeStruct((BATCH, 2 * DIM), F32)

    @functools.partial(
        pl.kernel, mesh=mesh,
        out_type=(out,) * 4,
        scratch_types=[pltpu.VMEM((bpw,), jnp.int32),
                       pltpu.VMEM((bpw, 2 * DIM), F32),
                       pltpu.SemaphoreType.DMA],
    )
    def k(tu_hbm, ti_hbm, rid_hbm, cid_hbm, pos_hbm, neg_hbm,
          o_ru, o_rc, o_rp, o_rn, idx_v, rows_v, sem):
        wid = lax.axis_index("s") * nc + lax.axis_index("c")
        base = wid * bpw

        def gather(idx_hbm, table_hbm, out_hbm):
            pltpu.sync_copy(idx_hbm.at[pl.ds(base, bpw)], idx_v)
            pltpu.async_copy(table_hbm.at[idx_v], rows_v, sem).wait()
            pltpu.sync_copy(rows_v, out_hbm.at[pl.ds(base, bpw)])

        gather(rid_hbm, tu_hbm, o_ru)
        gather(cid_hbm, ti_hbm, o_rc)
        gather(pos_hbm, ti_hbm, o_rp)
        gather(neg_hbm, ti_hbm, o_rn)

    return k(ue_u, ue_i, row_ids, col_ids, pos, neg)


_BB = 512
_NB = BATCH // _BB
_UCH = 2000
_ICH = 2500


def _loss_body(ueu_ref, eit_ref, reg_ref, ru_ref, rc_ref, rp_ref, rn_ref,
               loss_ref, oth_ref, acc_ref):
    i = pl.program_id(0)
    gur, eur = ru_ref[:, :DIM], ru_ref[:, DIM:]
    gic, eic = rc_ref[:, :DIM], rc_ref[:, DIM:]
    eip, ein = rp_ref[:, DIM:], rn_ref[:, DIM:]

    su = jnp.zeros((_BB, 1), F32)
    for kc in range(N_U // _UCH):
        logits = _dot_t1(gur, ueu_ref[kc * _UCH:(kc + 1) * _UCH, DIM:])
        su = su + jnp.sum(jnp.exp(logits * (1.0 / TEMP)), axis=1, keepdims=True)
    si = jnp.zeros((_BB, 1), F32)
    for kc in range(N_I // _ICH):
        logits = _dot(gic, eit_ref[:, kc * _ICH:(kc + 1) * _ICH])
        si = si + jnp.sum(jnp.exp(logits * (1.0 / TEMP)), axis=1, keepdims=True)
    nl_u = jnp.sum(jnp.log(su + 1e-08))
    nl_i = jnp.sum(jnp.log(si + 1e-08))

    pu = jnp.sum(gur * eur, axis=1, keepdims=True) * (1.0 / TEMP)
    pi = jnp.sum(gic * eic, axis=1, keepdims=True) * (1.0 / TEMP)
    pos = jnp.sum(jnp.clip(pu, -5.0, 5.0)) + jnp.sum(jnp.clip(pi, -5.0, 5.0))
    d = jnp.sum(eur * eip, axis=1, keepdims=True) - \
        jnp.sum(eur * ein, axis=1, keepdims=True)
    bpr = jnp.sum(jnp.log(1.0 + jnp.exp(-d)))

    upd = jnp.concatenate(
        [jnp.reshape(nl_u, (1, 1)), jnp.reshape(nl_i, (1, 1)),
         jnp.reshape(pos, (1, 1)), jnp.reshape(bpr, (1, 1)),
         jnp.zeros((1, 124), F32)], axis=1)

    @pl.when(i == 0)
    def _():
        acc_ref[...] = jnp.zeros_like(acc_ref)
    acc_ref[...] += upd

    @pl.when(i == _NB - 1)
    def _():
        acc = acc_ref[...]
        inv_b = 1.0 / BATCH
        neg_score = (acc[0, 0] + acc[0, 1]) * inv_b
        pos_score = acc[0, 2] * inv_b
        loss_bpr = acc[0, 3] * inv_b
        loss_cl = -pos_score + neg_score
        loss = loss_bpr + LAMBDA_1 * loss_cl + LAMBDA_2 * reg_ref[0, 0]
        loss_ref[...] = jnp.reshape(loss, (1, 1))
        oth_ref[...] = jnp.concatenate(
            [jnp.full((1, 1), loss_bpr, F32),
             jnp.full((1, 1), LAMBDA_1 * loss_cl, F32)], axis=1)


def _loss(ueu, eit, reg, ru, rc, rp, rn):
    bspec = pl.BlockSpec((_BB, 2 * DIM), lambda i: (i, 0))
    return pl.pallas_call(
        _loss_body,
        grid=(_NB,),
        in_specs=[pl.BlockSpec((N_U, 2 * DIM), lambda i: (0, 0)),
                  pl.BlockSpec((DIM, N_I), lambda i: (0, 0)),
                  pl.BlockSpec((1, 1), lambda i: (0, 0)),
                  bspec, bspec, bspec, bspec],
        out_specs=[pl.BlockSpec((1, 1), lambda i: (0, 0)),
                   pl.BlockSpec((1, 2), lambda i: (0, 0))],
        out_shape=[jax.ShapeDtypeStruct((1, 1), F32),
                   jax.ShapeDtypeStruct((1, 2), F32)],
        scratch_shapes=[pltpu.VMEM((1, 128), F32)],
    )(ueu, eit, reg, ru, rc, rp, rn)


def kernel(adj, row_ids, col_ids, pos, neg, E_u_0, E_i_0):
    g0t = jax.random.normal(jax.random.key(42), (N_I, SVD_Q), dtype=F32).T
    ei0t = E_i_0.T

    y0, c0, a16 = _pass_first(adj, g0t)
    x0 = _cholinv(c0)
    z1t = _chol_orth(_pass_transpose(a16, y0, x0))
    y1, c1 = _pass_fwd_gram(a16, z1t)
    x2 = _cholinv(c1)
    z2t = _chol_orth(_pass_transpose(a16, y1, x2))

    y2, c4, bt, zu1, zi1t, reg = _pass_five(a16, z2t, E_u_0, ei0t)
    m, t1 = _cholmt(c4, bt, ei0t, zi1t)
    ue_u, e_it, t2t = _pass_seven(a16, y2, t1, zu1, zi1t, E_u_0, ei0t)
    ue_i = _gi(bt, m, t2t, ei0t, e_it).T

    ru, rc, rp, rn = _sc_gather_all(ue_u, ue_i, row_ids, col_ids, pos, neg)

    loss, oth = _loss(ue_u, e_it, reg, ru, rc, rp, rn)
    return loss[0, 0], oth[0]


# speedup vs baseline: 3.5560x; 1.0408x over previous
import functools

import jax
import jax.numpy as jnp
from jax import lax
from jax.experimental import pallas as pl
from jax.experimental.pallas import tpu as pltpu
from jax.experimental.pallas import tpu_sc as plsc

N_U = 10000
N_I = 5000
DIM = 64
TEMP = 0.2
LAMBDA_1 = 0.2
LAMBDA_2 = 1e-07
SVD_Q = 32
BATCH = 4096

BM = 1000
GRID_U = N_U // BM
F32 = jnp.float32
_HI = jax.lax.Precision.HIGHEST


def _dot(a, b, precision=None):
    return jax.lax.dot_general(a, b, (((1,), (0,)), ((), ())),
                               precision=precision, preferred_element_type=F32)


def _dot_t0(a, b):
    return jax.lax.dot_general(a, b, (((0,), (0,)), ((), ())),
                               preferred_element_type=F32)


def _dot_t1(a, b):
    return jax.lax.dot_general(a, b, (((1,), (1,)), ((), ())),
                               preferred_element_type=F32)


BF16 = jnp.bfloat16


def _p0_body(a_ref, gt_ref, eu0_ref, ei0t_ref,
             y_ref, c_ref, a16_ref, zu1_ref, zi1t_ref, reg_ref):
    a = a_ref[...]
    a16_ref[...] = a.astype(BF16)
    y = _dot_t1(a_ref[...], gt_ref[...])
    y_ref[...] = y
    zu1_ref[...] = _dot_t1(a_ref[...], ei0t_ref[...])

    @pl.when(pl.program_id(0) == 0)
    def _():
        c_ref[...] = jnp.zeros_like(c_ref)
        zi1t_ref[...] = jnp.zeros_like(zi1t_ref)
        reg_ref[...] = jnp.reshape(
            jnp.sum(ei0t_ref[...] * ei0t_ref[...]), (1, 1))

    c_ref[...] += _dot_t0(y, y)
    zi1t_ref[...] += _dot_t0(eu0_ref[...], a_ref[...])
    reg_ref[...] += jnp.reshape(jnp.sum(eu0_ref[...] * eu0_ref[...]), (1, 1))


def _pass_first(adj, gt, eu0, ei0t):
    kq = gt.shape[0]
    bm0 = 400
    return pl.pallas_call(
        _p0_body,
        grid=(N_U // bm0,),
        in_specs=[pl.BlockSpec((bm0, N_I), lambda i: (i, 0)),
                  pl.BlockSpec((kq, N_I), lambda i: (0, 0)),
                  pl.BlockSpec((bm0, DIM), lambda i: (i, 0)),
                  pl.BlockSpec((DIM, N_I), lambda i: (0, 0))],
        out_specs=[pl.BlockSpec((bm0, kq), lambda i: (i, 0)),
                   pl.BlockSpec((kq, kq), lambda i: (0, 0)),
                   pl.BlockSpec((bm0, N_I), lambda i: (i, 0)),
                   pl.BlockSpec((bm0, DIM), lambda i: (i, 0)),
                   pl.BlockSpec((DIM, N_I), lambda i: (0, 0)),
                   pl.BlockSpec((1, 1), lambda i: (0, 0))],
        out_shape=[jax.ShapeDtypeStruct((N_U, kq), F32),
                   jax.ShapeDtypeStruct((kq, kq), F32),
                   jax.ShapeDtypeStruct((N_U, N_I), BF16),
                   jax.ShapeDtypeStruct((N_U, DIM), F32),
                   jax.ShapeDtypeStruct((DIM, N_I), F32),
                   jax.ShapeDtypeStruct((1, 1), F32)],
    )(adj, gt, eu0, ei0t)


def _p1_body(a_ref, gt_ref, y_ref, c_ref):
    y = _dot_t1(a_ref[...], gt_ref[...].astype(BF16))
    y_ref[...] = y

    @pl.when(pl.program_id(0) == 0)
    def _():
        c_ref[...] = jnp.zeros_like(c_ref)
    c_ref[...] += _dot_t0(y, y)


def _pass_fwd_gram(a16, gt):
    kq = gt.shape[0]
    return pl.pallas_call(
        _p1_body,
        grid=(GRID_U,),
        in_specs=[pl.BlockSpec((BM, N_I), lambda i: (i, 0)),
                  pl.BlockSpec((kq, N_I), lambda i: (0, 0))],
        out_specs=[pl.BlockSpec((BM, kq), lambda i: (i, 0)),
                   pl.BlockSpec((kq, kq), lambda i: (0, 0))],
        out_shape=[jax.ShapeDtypeStruct((N_U, kq), F32),
                   jax.ShapeDtypeStruct((kq, kq), F32)],
    )(a16, gt)


def _p2_body(a_ref, w_ref, x_ref, o_ref):
    q = _dot_t1(w_ref[...], x_ref[...])

    @pl.when(pl.program_id(0) == 0)
    def _():
        o_ref[...] = jnp.zeros_like(o_ref)
    o_ref[...] += _dot_t0(q.astype(BF16), a_ref[...])


def _pass_transpose(adj, w, x):
    kq = w.shape[1]
    return pl.pallas_call(
        _p2_body,
        grid=(GRID_U,),
        in_specs=[pl.BlockSpec((BM, N_I), lambda i: (i, 0)),
                  pl.BlockSpec((BM, kq), lambda i: (i, 0)),
                  pl.BlockSpec((kq, kq), lambda i: (0, 0))],
        out_specs=pl.BlockSpec((kq, N_I), lambda i: (0, 0)),
        out_shape=jax.ShapeDtypeStruct((kq, N_I), F32),
    )(adj, w, x)


def _cholinv_math(C):
    q = SVD_Q
    ri = jax.lax.broadcasted_iota(jnp.int32, (q, q), 0)
    ci = jax.lax.broadcasted_iota(jnp.int32, (q, q), 1)
    ri1 = jax.lax.broadcasted_iota(jnp.int32, (q, 1), 0)
    ci1 = jax.lax.broadcasted_iota(jnp.int32, (1, q), 1)

    def chol_step(j, carry):
        L, Ck = carry
        dj = jnp.sum(jnp.where((ri == j) & (ci == j), Ck, 0.0))
        inv_s = jax.lax.rsqrt(dj)
        col = jnp.sum(jnp.where(ci == j, Ck, 0.0), axis=1, keepdims=True)
        col = jnp.where(ri1 >= j, col, 0.0) * inv_s
        row = jnp.sum(jnp.where(ri == j, Ck, 0.0), axis=0, keepdims=True)
        row = jnp.where(ci1 >= j, row, 0.0) * inv_s
        L = L + jnp.where(ci == j, col, 0.0)
        Ck = Ck - col * row
        return L, Ck

    L, _ = jax.lax.fori_loop(0, q, chol_step, (jnp.zeros_like(C), C))

    def inv_step(i, X):
        lrow = jnp.sum(jnp.where(ri == i, L, 0.0), axis=0, keepdims=True)
        dii = jnp.sum(jnp.where(ci1 == i, lrow, 0.0))
        lrow = jnp.where(ci1 < i, lrow, 0.0)
        prod = _dot(lrow, X, precision=_HI)
        xrow = (jnp.where(ci1 == i, 1.0, 0.0) - prod) / dii
        return X + jnp.where(ri == i, xrow, 0.0)

    return jax.lax.fori_loop(0, q, inv_step, jnp.zeros_like(C))


def _cholinv_body(c_ref, x_ref):
    x_ref[...] = _cholinv_math(c_ref[...])


def _cholinv(c):
    kq = c.shape[0]
    return pl.pallas_call(
        _cholinv_body,
        out_shape=jax.ShapeDtypeStruct((kq, kq), F32),
    )(c)


def _chol_orth_body(wt_ref, zt_ref):
    wt = wt_ref[...]
    x = _cholinv_math(_dot_t1(wt, wt))
    zt_ref[...] = _dot(x, wt)


def _chol_orth(wt):
    kq, n = wt.shape
    return pl.pallas_call(
        _chol_orth_body,
        out_shape=jax.ShapeDtypeStruct((kq, n), F32),
    )(wt)


def _p5_body(a_ref, zt_ref, zu1_ref, zi1t_ref, eu0_ref, ei0t_ref,
             y_ref, c_ref, bt_ref, eu_ref, eit_ref, t2t_ref):
    y = _dot_t1(a_ref[...], zt_ref[...].astype(BF16))
    y_ref[...] = y
    zu1 = zu1_ref[...]
    eu0 = eu0_ref[...]
    eu_ref[...] = eu0 + zu1 + _dot_t1(a_ref[...], zi1t_ref[...].astype(BF16))

    @pl.when(pl.program_id(0) == 0)
    def _():
        c_ref[...] = jnp.zeros_like(c_ref)
        bt_ref[...] = jnp.zeros_like(bt_ref)
        eit_ref[...] = ei0t_ref[...] + zi1t_ref[...]
        t2t_ref[...] = jnp.zeros_like(t2t_ref)

    c_ref[...] += _dot_t0(y, y)
    bt_ref[...] += _dot_t0(y.astype(BF16), a_ref[...])
    eit_ref[...] += _dot_t0(zu1.astype(BF16), a_ref[...])
    t2t_ref[...] += _dot_t0(eu0 + zu1, y)


def _pass_final(a16, zt, zu1, zi1t, eu0, ei0t):
    return pl.pallas_call(
        _p5_body,
        grid=(GRID_U,),
        in_specs=[pl.BlockSpec((BM, N_I), lambda i: (i, 0)),
                  pl.BlockSpec((SVD_Q, N_I), lambda i: (0, 0)),
                  pl.BlockSpec((BM, DIM), lambda i: (i, 0)),
                  pl.BlockSpec((DIM, N_I), lambda i: (0, 0)),
                  pl.BlockSpec((BM, DIM), lambda i: (i, 0)),
                  pl.BlockSpec((DIM, N_I), lambda i: (0, 0))],
        out_specs=[pl.BlockSpec((BM, SVD_Q), lambda i: (i, 0)),
                   pl.BlockSpec((SVD_Q, SVD_Q), lambda i: (0, 0)),
                   pl.BlockSpec((SVD_Q, N_I), lambda i: (0, 0)),
                   pl.BlockSpec((BM, DIM), lambda i: (i, 0)),
                   pl.BlockSpec((DIM, N_I), lambda i: (0, 0)),
                   pl.BlockSpec((DIM, SVD_Q), lambda i: (0, 0))],
        out_shape=[jax.ShapeDtypeStruct((N_U, SVD_Q), F32),
                   jax.ShapeDtypeStruct((SVD_Q, SVD_Q), F32),
                   jax.ShapeDtypeStruct((SVD_Q, N_I), F32),
                   jax.ShapeDtypeStruct((N_U, DIM), F32),
                   jax.ShapeDtypeStruct((DIM, N_I), F32),
                   jax.ShapeDtypeStruct((DIM, SVD_Q), F32)],
    )(a16, zt, zu1, zi1t, eu0, ei0t)


def _cholmt_body(c_ref, bt_ref, ei0t_ref, zi1t_ref, m_ref, t1_ref):
    x = _cholinv_math(c_ref[...])
    m = _dot_t0(x, x)
    m_ref[...] = m
    t1_ref[...] = _dot(m, _dot_t1(bt_ref[...], ei0t_ref[...] + zi1t_ref[...]))


def _cholmt(c, bt, ei0t, zi1t):
    return pl.pallas_call(
        _cholmt_body,
        out_shape=[jax.ShapeDtypeStruct((SVD_Q, SVD_Q), F32),
                   jax.ShapeDtypeStruct((SVD_Q, DIM), F32)],
    )(c, bt, ei0t, zi1t)


def _guei_body(y_ref, t1_ref, eu0_ref, eu_ref, ueu_ref):
    ueu_ref[...] = jnp.concatenate(
        [eu0_ref[...] + _dot(y_ref[...], t1_ref[...]), eu_ref[...]], axis=1)


def _guei(y2, t1, eu0, e_u):
    return pl.pallas_call(
        _guei_body,
        grid=(GRID_U,),
        in_specs=[pl.BlockSpec((BM, SVD_Q), lambda i: (i, 0)),
                  pl.BlockSpec((SVD_Q, DIM), lambda i: (0, 0)),
                  pl.BlockSpec((BM, DIM), lambda i: (i, 0)),
                  pl.BlockSpec((BM, DIM), lambda i: (i, 0))],
        out_specs=pl.BlockSpec((BM, 2 * DIM), lambda i: (i, 0)),
        out_shape=jax.ShapeDtypeStruct((N_U, 2 * DIM), F32),
    )(y2, t1, eu0, e_u)


def _gi_body(bt_ref, m_ref, t2t_ref, ei0t_ref, eit_ref, out_ref):
    git = ei0t_ref[...] + _dot(_dot(t2t_ref[...], m_ref[...]), bt_ref[...])
    out_ref[...] = jnp.concatenate([git, eit_ref[...]], axis=0)


def _gi(bt, m, t2t, ei0t, eit):
    return pl.pallas_call(
        _gi_body,
        out_shape=jax.ShapeDtypeStruct((2 * DIM, N_I), F32),
    )(bt, m, t2t, ei0t, eit)


def _sc_gather_all(ue_u, ue_i, row_ids, col_ids, pos, neg):
    info = plsc.get_sparse_core_info()
    nc, ns = info.num_cores, info.num_subcores
    nw = nc * ns
    bpw = BATCH // nw
    mesh = plsc.VectorSubcoreMesh(core_axis_name="c", subcore_axis_name="s")
    out = jax.ShapeDtypeStruct((BATCH, 2 * DIM), F32)

    @functools.partial(
        pl.kernel, mesh=mesh,
        out_type=(out,) * 4,
        scratch_types=[pltpu.VMEM((bpw,), jnp.int32),
                       pltpu.VMEM((bpw, 2 * DIM), F32),
                       pltpu.SemaphoreType.DMA],
    )
    def k(tu_hbm, ti_hbm, rid_hbm, cid_hbm, pos_hbm, neg_hbm,
          o_ru, o_rc, o_rp, o_rn, idx_v, rows_v, sem):
        wid = lax.axis_index("s") * nc + lax.axis_index("c")
        base = wid * bpw

        def gather(idx_hbm, table_hbm, out_hbm):
            pltpu.sync_copy(idx_hbm.at[pl.ds(base, bpw)], idx_v)
            pltpu.async_copy(table_hbm.at[idx_v], rows_v, sem).wait()
            pltpu.sync_copy(rows_v, out_hbm.at[pl.ds(base, bpw)])

        gather(rid_hbm, tu_hbm, o_ru)
        gather(cid_hbm, ti_hbm, o_rc)
        gather(pos_hbm, ti_hbm, o_rp)
        gather(neg_hbm, ti_hbm, o_rn)

    return k(ue_u, ue_i, row_ids, col_ids, pos, neg)


_BB = 512
_NB = BATCH // _BB
_UCH = 2000
_ICH = 2500


def _loss_body(ueu_ref, eit_ref, reg_ref, ru_ref, rc_ref, rp_ref, rn_ref,
               loss_ref, oth_ref, acc_ref):
    i = pl.program_id(0)
    gur, eur = ru_ref[:, :DIM], ru_ref[:, DIM:]
    gic, eic = rc_ref[:, :DIM], rc_ref[:, DIM:]
    eip, ein = rp_ref[:, DIM:], rn_ref[:, DIM:]

    su = jnp.zeros((_BB, 1), F32)
    for kc in range(N_U // _UCH):
        logits = _dot_t1(gur, ueu_ref[kc * _UCH:(kc + 1) * _UCH, DIM:])
        su = su + jnp.sum(jnp.exp(logits * (1.0 / TEMP)), axis=1, keepdims=True)
    si = jnp.zeros((_BB, 1), F32)
    for kc in range(N_I // _ICH):
        logits = _dot(gic, eit_ref[:, kc * _ICH:(kc + 1) * _ICH])
        si = si + jnp.sum(jnp.exp(logits * (1.0 / TEMP)), axis=1, keepdims=True)
    nl_u = jnp.sum(jnp.log(su + 1e-08))
    nl_i = jnp.sum(jnp.log(si + 1e-08))

    pu = jnp.sum(gur * eur, axis=1, keepdims=True) * (1.0 / TEMP)
    pi = jnp.sum(gic * eic, axis=1, keepdims=True) * (1.0 / TEMP)
    pos = jnp.sum(jnp.clip(pu, -5.0, 5.0)) + jnp.sum(jnp.clip(pi, -5.0, 5.0))
    d = jnp.sum(eur * eip, axis=1, keepdims=True) - \
        jnp.sum(eur * ein, axis=1, keepdims=True)
    bpr = jnp.sum(jnp.log(1.0 + jnp.exp(-d)))

    upd = jnp.concatenate(
        [jnp.reshape(nl_u, (1, 1)), jnp.reshape(nl_i, (1, 1)),
         jnp.reshape(pos, (1, 1)), jnp.reshape(bpr, (1, 1)),
         jnp.zeros((1, 124), F32)], axis=1)

    @pl.when(i == 0)
    def _():
        acc_ref[...] = jnp.zeros_like(acc_ref)
    acc_ref[...] += upd

    @pl.when(i == _NB - 1)
    def _():
        acc = acc_ref[...]
        inv_b = 1.0 / BATCH
        neg_score = (acc[0, 0] + acc[0, 1]) * inv_b
        pos_score = acc[0, 2] * inv_b
        loss_bpr = acc[0, 3] * inv_b
        loss_cl = -pos_score + neg_score
        loss = loss_bpr + LAMBDA_1 * loss_cl + LAMBDA_2 * reg_ref[0, 0]
        loss_ref[...] = jnp.reshape(loss, (1, 1))
        oth_ref[...] = jnp.concatenate(
            [jnp.full((1, 1), loss_bpr, F32),
             jnp.full((1, 1), LAMBDA_1 * loss_cl, F32)], axis=1)


def _loss(ueu, eit, reg, ru, rc, rp, rn):
    bspec = pl.BlockSpec((_BB, 2 * DIM), lambda i: (i, 0))
    return pl.pallas_call(
        _loss_body,
        grid=(_NB,),
        in_specs=[pl.BlockSpec((N_U, 2 * DIM), lambda i: (0, 0)),
                  pl.BlockSpec((DIM, N_I), lambda i: (0, 0)),
                  pl.BlockSpec((1, 1), lambda i: (0, 0)),
                  bspec, bspec, bspec, bspec],
        out_specs=[pl.BlockSpec((1, 1), lambda i: (0, 0)),
                   pl.BlockSpec((1, 2), lambda i: (0, 0))],
        out_shape=[jax.ShapeDtypeStruct((1, 1), F32),
                   jax.ShapeDtypeStruct((1, 2), F32)],
        scratch_shapes=[pltpu.VMEM((1, 128), F32)],
    )(ueu, eit, reg, ru, rc, rp, rn)


def kernel(adj, row_ids, col_ids, pos, neg, E_u_0, E_i_0):
    g0t = jax.random.normal(jax.random.key(42), (N_I, SVD_Q), dtype=F32).T
    ei0t = E_i_0.T

    y0, c0, a16, zu1, zi1t, reg = _pass_first(adj, g0t, E_u_0, ei0t)
    x0 = _cholinv(c0)
    z1t = _chol_orth(_pass_transpose(a16, y0, x0))
    y1, c1 = _pass_fwd_gram(a16, z1t)
    x2 = _cholinv(c1)
    z2t = _chol_orth(_pass_transpose(a16, y1, x2))
    y2, c4, bt, e_u, e_it, t2t = _pass_final(a16, z2t, zu1, zi1t, E_u_0, ei0t)
    m, t1 = _cholmt(c4, bt, ei0t, zi1t)
    ue_u = _guei(y2, t1, E_u_0, e_u)
    ue_i = _gi(bt, m, t2t, ei0t, e_it).T

    ru, rc, rp, rn = _sc_gather_all(ue_u, ue_i, row_ids, col_ids, pos, neg)

    loss, oth = _loss(ue_u, e_it, reg, ru, rc, rp, rn)
    return loss[0, 0], oth[0]


# speedup vs baseline: 3.6269x; 1.0199x over previous
import functools

import jax
import jax.numpy as jnp
from jax import lax
from jax.experimental import pallas as pl
from jax.experimental.pallas import tpu as pltpu
from jax.experimental.pallas import tpu_sc as plsc

N_U = 10000
N_I = 5000
DIM = 64
TEMP = 0.2
LAMBDA_1 = 0.2
LAMBDA_2 = 1e-07
SVD_Q = 32
BATCH = 4096

BM = 1000
GRID_U = N_U // BM
F32 = jnp.float32
_HI = jax.lax.Precision.HIGHEST


def _dot(a, b, precision=None):
    return jax.lax.dot_general(a, b, (((1,), (0,)), ((), ())),
                               precision=precision, preferred_element_type=F32)


def _dot_t0(a, b):
    return jax.lax.dot_general(a, b, (((0,), (0,)), ((), ())),
                               preferred_element_type=F32)


def _dot_t1(a, b):
    return jax.lax.dot_general(a, b, (((1,), (1,)), ((), ())),
                               preferred_element_type=F32)


BF16 = jnp.bfloat16


def _p0_body(a_ref, gt_ref, eu0_ref, ei0t_ref,
             y_ref, c_ref, a16_ref, zu1_ref, zi1t_ref, reg_ref):
    a = a_ref[...]
    a16_ref[...] = a.astype(BF16)
    y = _dot_t1(a_ref[...], gt_ref[...])
    y_ref[...] = y
    zu1_ref[...] = _dot_t1(a_ref[...], ei0t_ref[...])

    @pl.when(pl.program_id(0) == 0)
    def _():
        c_ref[...] = jnp.zeros_like(c_ref)
        zi1t_ref[...] = jnp.zeros_like(zi1t_ref)
        reg_ref[...] = jnp.reshape(
            jnp.sum(ei0t_ref[...] * ei0t_ref[...]), (1, 1))

    c_ref[...] += _dot_t0(y, y)
    zi1t_ref[...] += _dot_t0(eu0_ref[...], a_ref[...])
    reg_ref[...] += jnp.reshape(jnp.sum(eu0_ref[...] * eu0_ref[...]), (1, 1))


def _pass_first(adj, gt, eu0, ei0t):
    kq = gt.shape[0]
    bm0 = 400
    return pl.pallas_call(
        _p0_body,
        grid=(N_U // bm0,),
        in_specs=[pl.BlockSpec((bm0, N_I), lambda i: (i, 0)),
                  pl.BlockSpec((kq, N_I), lambda i: (0, 0)),
                  pl.BlockSpec((bm0, DIM), lambda i: (i, 0)),
                  pl.BlockSpec((DIM, N_I), lambda i: (0, 0))],
        out_specs=[pl.BlockSpec((bm0, kq), lambda i: (i, 0)),
                   pl.BlockSpec((kq, kq), lambda i: (0, 0)),
                   pl.BlockSpec((bm0, N_I), lambda i: (i, 0)),
                   pl.BlockSpec((bm0, DIM), lambda i: (i, 0)),
                   pl.BlockSpec((DIM, N_I), lambda i: (0, 0)),
                   pl.BlockSpec((1, 1), lambda i: (0, 0))],
        out_shape=[jax.ShapeDtypeStruct((N_U, kq), F32),
                   jax.ShapeDtypeStruct((kq, kq), F32),
                   jax.ShapeDtypeStruct((N_U, N_I), BF16),
                   jax.ShapeDtypeStruct((N_U, DIM), F32),
                   jax.ShapeDtypeStruct((DIM, N_I), F32),
                   jax.ShapeDtypeStruct((1, 1), F32)],
    )(adj, gt, eu0, ei0t)


def _p1_body(a_ref, wt_ref, y_ref, c_ref, zt_ref):
    @pl.when(pl.program_id(0) == 0)
    def _():
        wt = wt_ref[...]
        x = _cholinv_math(_dot_t1(wt, wt))
        zt_ref[...] = _dot(x, wt).astype(BF16)
        c_ref[...] = jnp.zeros_like(c_ref)

    y = _dot_t1(a_ref[...], zt_ref[...])
    y_ref[...] = y
    c_ref[...] += _dot_t0(y, y)


def _pass_fwd_gram(a16, wt):
    kq = wt.shape[0]
    return pl.pallas_call(
        _p1_body,
        grid=(GRID_U,),
        in_specs=[pl.BlockSpec((BM, N_I), lambda i: (i, 0)),
                  pl.BlockSpec((kq, N_I), lambda i: (0, 0))],
        out_specs=[pl.BlockSpec((BM, kq), lambda i: (i, 0)),
                   pl.BlockSpec((kq, kq), lambda i: (0, 0))],
        out_shape=[jax.ShapeDtypeStruct((N_U, kq), F32),
                   jax.ShapeDtypeStruct((kq, kq), F32)],
        scratch_shapes=[pltpu.VMEM((kq, N_I), BF16)],
    )(a16, wt)


def _p2_body(a_ref, w_ref, c_ref, o_ref, x_ref):
    @pl.when(pl.program_id(0) == 0)
    def _():
        x_ref[...] = _cholinv_math(c_ref[...])
        o_ref[...] = jnp.zeros_like(o_ref)

    q = _dot_t1(w_ref[...], x_ref[...])
    o_ref[...] += _dot_t0(q.astype(BF16), a_ref[...])


def _pass_transpose(a16, w, c):
    kq = w.shape[1]
    return pl.pallas_call(
        _p2_body,
        grid=(GRID_U,),
        in_specs=[pl.BlockSpec((BM, N_I), lambda i: (i, 0)),
                  pl.BlockSpec((BM, kq), lambda i: (i, 0)),
                  pl.BlockSpec((kq, kq), lambda i: (0, 0))],
        out_specs=pl.BlockSpec((kq, N_I), lambda i: (0, 0)),
        out_shape=jax.ShapeDtypeStruct((kq, N_I), F32),
        scratch_shapes=[pltpu.VMEM((kq, kq), F32)],
    )(a16, w, c)


def _cholinv_math(C):
    q = SVD_Q
    ri = jax.lax.broadcasted_iota(jnp.int32, (q, q), 0)
    ci = jax.lax.broadcasted_iota(jnp.int32, (q, q), 1)
    ri1 = jax.lax.broadcasted_iota(jnp.int32, (q, 1), 0)
    ci1 = jax.lax.broadcasted_iota(jnp.int32, (1, q), 1)

    def chol_step(j, carry):
        L, Ck = carry
        dj = jnp.sum(jnp.where((ri == j) & (ci == j), Ck, 0.0))
        inv_s = jax.lax.rsqrt(dj)
        col = jnp.sum(jnp.where(ci == j, Ck, 0.0), axis=1, keepdims=True)
        col = jnp.where(ri1 >= j, col, 0.0) * inv_s
        row = jnp.sum(jnp.where(ri == j, Ck, 0.0), axis=0, keepdims=True)
        row = jnp.where(ci1 >= j, row, 0.0) * inv_s
        L = L + jnp.where(ci == j, col, 0.0)
        Ck = Ck - col * row
        return L, Ck

    L, _ = jax.lax.fori_loop(0, q, chol_step, (jnp.zeros_like(C), C))

    def inv_step(i, X):
        lrow = jnp.sum(jnp.where(ri == i, L, 0.0), axis=0, keepdims=True)
        dii = jnp.sum(jnp.where(ci1 == i, lrow, 0.0))
        lrow = jnp.where(ci1 < i, lrow, 0.0)
        prod = _dot(lrow, X, precision=_HI)
        xrow = (jnp.where(ci1 == i, 1.0, 0.0) - prod) / dii
        return X + jnp.where(ri == i, xrow, 0.0)

    return jax.lax.fori_loop(0, q, inv_step, jnp.zeros_like(C))


def _p5_body(a_ref, wt_ref, zu1_ref, zi1t_ref, eu0_ref, ei0t_ref,
             y_ref, c_ref, bt_ref, eu_ref, eit_ref, t2t_ref, zt_ref):
    @pl.when(pl.program_id(0) == 0)
    def _():
        wt = wt_ref[...]
        x = _cholinv_math(_dot_t1(wt, wt))
        zt_ref[...] = _dot(x, wt).astype(BF16)
        c_ref[...] = jnp.zeros_like(c_ref)
        bt_ref[...] = jnp.zeros_like(bt_ref)
        eit_ref[...] = ei0t_ref[...] + zi1t_ref[...]
        t2t_ref[...] = jnp.zeros_like(t2t_ref)

    y = _dot_t1(a_ref[...], zt_ref[...])
    y_ref[...] = y
    zu1 = zu1_ref[...]
    eu0 = eu0_ref[...]
    eu_ref[...] = eu0 + zu1 + _dot_t1(a_ref[...], zi1t_ref[...].astype(BF16))

    c_ref[...] += _dot_t0(y, y)
    bt_ref[...] += _dot_t0(y.astype(BF16), a_ref[...])
    eit_ref[...] += _dot_t0(zu1.astype(BF16), a_ref[...])
    t2t_ref[...] += _dot_t0(eu0 + zu1, y)


def _pass_final(a16, wt, zu1, zi1t, eu0, ei0t):
    return pl.pallas_call(
        _p5_body,
        grid=(GRID_U,),
        in_specs=[pl.BlockSpec((BM, N_I), lambda i: (i, 0)),
                  pl.BlockSpec((SVD_Q, N_I), lambda i: (0, 0)),
                  pl.BlockSpec((BM, DIM), lambda i: (i, 0)),
                  pl.BlockSpec((DIM, N_I), lambda i: (0, 0)),
                  pl.BlockSpec((BM, DIM), lambda i: (i, 0)),
                  pl.BlockSpec((DIM, N_I), lambda i: (0, 0))],
        out_specs=[pl.BlockSpec((BM, SVD_Q), lambda i: (i, 0)),
                   pl.BlockSpec((SVD_Q, SVD_Q), lambda i: (0, 0)),
                   pl.BlockSpec((SVD_Q, N_I), lambda i: (0, 0)),
                   pl.BlockSpec((BM, DIM), lambda i: (i, 0)),
                   pl.BlockSpec((DIM, N_I), lambda i: (0, 0)),
                   pl.BlockSpec((DIM, SVD_Q), lambda i: (0, 0))],
        out_shape=[jax.ShapeDtypeStruct((N_U, SVD_Q), F32),
                   jax.ShapeDtypeStruct((SVD_Q, SVD_Q), F32),
                   jax.ShapeDtypeStruct((SVD_Q, N_I), F32),
                   jax.ShapeDtypeStruct((N_U, DIM), F32),
                   jax.ShapeDtypeStruct((DIM, N_I), F32),
                   jax.ShapeDtypeStruct((DIM, SVD_Q), F32)],
        scratch_shapes=[pltpu.VMEM((SVD_Q, N_I), BF16)],
    )(a16, wt, zu1, zi1t, eu0, ei0t)


def _guei_body(y_ref, c_ref, bt_ref, t2t_ref, eu0_ref, eu_ref,
               ei0t_ref, zi1t_ref, eit_ref, ueu_ref, uei_ref, t1_ref):
    @pl.when(pl.program_id(0) == 0)
    def _():
        x = _cholinv_math(c_ref[...])
        m = _dot_t0(x, x)
        t1_ref[...] = _dot(
            m, _dot_t1(bt_ref[...], ei0t_ref[...] + zi1t_ref[...]))
        git = ei0t_ref[...] + _dot(_dot(t2t_ref[...], m), bt_ref[...])
        uei_ref[...] = jnp.concatenate([git, eit_ref[...]], axis=0)

    ueu_ref[...] = jnp.concatenate(
        [eu0_ref[...] + _dot(y_ref[...], t1_ref[...]), eu_ref[...]], axis=1)


def _guei(y2, c4, bt, t2t, eu0, e_u, ei0t, zi1t, eit):
    return pl.pallas_call(
        _guei_body,
        grid=(GRID_U,),
        in_specs=[pl.BlockSpec((BM, SVD_Q), lambda i: (i, 0)),
                  pl.BlockSpec((SVD_Q, SVD_Q), lambda i: (0, 0)),
                  pl.BlockSpec((SVD_Q, N_I), lambda i: (0, 0)),
                  pl.BlockSpec((DIM, SVD_Q), lambda i: (0, 0)),
                  pl.BlockSpec((BM, DIM), lambda i: (i, 0)),
                  pl.BlockSpec((BM, DIM), lambda i: (i, 0)),
                  pl.BlockSpec((DIM, N_I), lambda i: (0, 0)),
                  pl.BlockSpec((DIM, N_I), lambda i: (0, 0)),
                  pl.BlockSpec((DIM, N_I), lambda i: (0, 0))],
        out_specs=[pl.BlockSpec((BM, 2 * DIM), lambda i: (i, 0)),
                   pl.BlockSpec((2 * DIM, N_I), lambda i: (0, 0))],
        out_shape=[jax.ShapeDtypeStruct((N_U, 2 * DIM), F32),
                   jax.ShapeDtypeStruct((2 * DIM, N_I), F32)],
        scratch_shapes=[pltpu.VMEM((SVD_Q, DIM), F32)],
    )(y2, c4, bt, t2t, eu0, e_u, ei0t, zi1t, eit)


def _sc_gather_all(ue_u, ue_i, row_ids, col_ids, pos, neg):
    info = plsc.get_sparse_core_info()
    nc, ns = info.num_cores, info.num_subcores
    nw = nc * ns
    bpw = BATCH // nw
    mesh = plsc.VectorSubcoreMesh(core_axis_name="c", subcore_axis_name="s")
    out = jax.ShapeDtypeStruct((BATCH, 2 * DIM), F32)

    @functools.partial(
        pl.kernel, mesh=mesh,
        out_type=(out,) * 4,
        scratch_types=[pltpu.VMEM((bpw,), jnp.int32),
                       pltpu.VMEM((bpw, 2 * DIM), F32),
                       pltpu.SemaphoreType.DMA],
    )
    def k(tu_hbm, ti_hbm, rid_hbm, cid_hbm, pos_hbm, neg_hbm,
          o_ru, o_rc, o_rp, o_rn, idx_v, rows_v, sem):
        wid = lax.axis_index("s") * nc + lax.axis_index("c")
        base = wid * bpw

        def gather(idx_hbm, table_hbm, out_hbm):
            pltpu.sync_copy(idx_hbm.at[pl.ds(base, bpw)], idx_v)
            pltpu.async_copy(table_hbm.at[idx_v], rows_v, sem).wait()
            pltpu.sync_copy(rows_v, out_hbm.at[pl.ds(base, bpw)])

        gather(rid_hbm, tu_hbm, o_ru)
        gather(cid_hbm, ti_hbm, o_rc)
        gather(pos_hbm, ti_hbm, o_rp)
        gather(neg_hbm, ti_hbm, o_rn)

    return k(ue_u, ue_i, row_ids, col_ids, pos, neg)


_BB = 512
_NB = BATCH // _BB
_UCH = 2000
_ICH = 2500


def _loss_body(ueu_ref, eit_ref, reg_ref, ru_ref, rc_ref, rp_ref, rn_ref,
               loss_ref, oth_ref, acc_ref):
    i = pl.program_id(0)
    gur, eur = ru_ref[:, :DIM], ru_ref[:, DIM:]
    gic, eic = rc_ref[:, :DIM], rc_ref[:, DIM:]
    eip, ein = rp_ref[:, DIM:], rn_ref[:, DIM:]

    su = jnp.zeros((_BB, 1), F32)
    for kc in range(N_U // _UCH):
        logits = _dot_t1(gur, ueu_ref[kc * _UCH:(kc + 1) * _UCH, DIM:])
        su = su + jnp.sum(jnp.exp(logits * (1.0 / TEMP)), axis=1, keepdims=True)
    si = jnp.zeros((_BB, 1), F32)
    for kc in range(N_I // _ICH):
        logits = _dot(gic, eit_ref[:, kc * _ICH:(kc + 1) * _ICH])
        si = si + jnp.sum(jnp.exp(logits * (1.0 / TEMP)), axis=1, keepdims=True)
    nl_u = jnp.sum(jnp.log(su + 1e-08))
    nl_i = jnp.sum(jnp.log(si + 1e-08))

    pu = jnp.sum(gur * eur, axis=1, keepdims=True) * (1.0 / TEMP)
    pi = jnp.sum(gic * eic, axis=1, keepdims=True) * (1.0 / TEMP)
    pos = jnp.sum(jnp.clip(pu, -5.0, 5.0)) + jnp.sum(jnp.clip(pi, -5.0, 5.0))
    d = jnp.sum(eur * eip, axis=1, keepdims=True) - \
        jnp.sum(eur * ein, axis=1, keepdims=True)
    bpr = jnp.sum(jnp.log(1.0 + jnp.exp(-d)))

    upd = jnp.concatenate(
        [jnp.reshape(nl_u, (1, 1)), jnp.reshape(nl_i, (1, 1)),
         jnp.reshape(pos, (1, 1)), jnp.reshape(bpr, (1, 1)),
         jnp.zeros((1, 124), F32)], axis=1)

    @pl.when(i == 0)
    def _():
        acc_ref[...] = jnp.zeros_like(acc_ref)
    acc_ref[...] += upd

    @pl.when(i == _NB - 1)
    def _():
        acc = acc_ref[...]
        inv_b = 1.0 / BATCH
        neg_score = (acc[0, 0] + acc[0, 1]) * inv_b
        pos_score = acc[0, 2] * inv_b
        loss_bpr = acc[0, 3] * inv_b
        loss_cl = -pos_score + neg_score
        loss = loss_bpr + LAMBDA_1 * loss_cl + LAMBDA_2 * reg_ref[0, 0]
        loss_ref[...] = jnp.reshape(loss, (1, 1))
        oth_ref[...] = jnp.concatenate(
            [jnp.full((1, 1), loss_bpr, F32),
             jnp.full((1, 1), LAMBDA_1 * loss_cl, F32)], axis=1)


def _loss(ueu, eit, reg, ru, rc, rp, rn):
    bspec = pl.BlockSpec((_BB, 2 * DIM), lambda i: (i, 0))
    return pl.pallas_call(
        _loss_body,
        grid=(_NB,),
        in_specs=[pl.BlockSpec((N_U, 2 * DIM), lambda i: (0, 0)),
                  pl.BlockSpec((DIM, N_I), lambda i: (0, 0)),
                  pl.BlockSpec((1, 1), lambda i: (0, 0)),
                  bspec, bspec, bspec, bspec],
        out_specs=[pl.BlockSpec((1, 1), lambda i: (0, 0)),
                   pl.BlockSpec((1, 2), lambda i: (0, 0))],
        out_shape=[jax.ShapeDtypeStruct((1, 1), F32),
                   jax.ShapeDtypeStruct((1, 2), F32)],
        scratch_shapes=[pltpu.VMEM((1, 128), F32)],
    )(ueu, eit, reg, ru, rc, rp, rn)


def kernel(adj, row_ids, col_ids, pos, neg, E_u_0, E_i_0):
    g0t = jax.random.normal(jax.random.key(42), (N_I, SVD_Q), dtype=F32).T
    ei0t = E_i_0.T

    y0, c0, a16, zu1, zi1t, reg = _pass_first(adj, g0t, E_u_0, ei0t)
    w1t = _pass_transpose(a16, y0, c0)
    y1, c1 = _pass_fwd_gram(a16, w1t)
    w2t = _pass_transpose(a16, y1, c1)
    y2, c4, bt, e_u, e_it, t2t = _pass_final(a16, w2t, zu1, zi1t, E_u_0, ei0t)
    ue_u, uei_t = _guei(y2, c4, bt, t2t, E_u_0, e_u, ei0t, zi1t, e_it)
    ue_i = uei_t.T

    ru, rc, rp, rn = _sc_gather_all(ue_u, ue_i, row_ids, col_ids, pos, neg)

    loss, oth = _loss(ue_u, e_it, reg, ru, rc, rp, rn)
    return loss[0, 0], oth[0]


# speedup vs baseline: 3.8486x; 1.0611x over previous
import functools

import jax
import jax.numpy as jnp
from jax import lax
from jax.experimental import pallas as pl
from jax.experimental.pallas import tpu as pltpu
from jax.experimental.pallas import tpu_sc as plsc

N_U = 10000
N_I = 5000
DIM = 64
TEMP = 0.2
LAMBDA_1 = 0.2
LAMBDA_2 = 1e-07
SVD_Q = 32
BATCH = 4096

BM = 1000
GRID_U = N_U // BM
F32 = jnp.float32
_HI = jax.lax.Precision.HIGHEST


def _dot(a, b, precision=None):
    return jax.lax.dot_general(a, b, (((1,), (0,)), ((), ())),
                               precision=precision, preferred_element_type=F32)


def _dot_t0(a, b):
    return jax.lax.dot_general(a, b, (((0,), (0,)), ((), ())),
                               preferred_element_type=F32)


def _dot_t1(a, b):
    return jax.lax.dot_general(a, b, (((1,), (1,)), ((), ())),
                               preferred_element_type=F32)


BF16 = jnp.bfloat16


def _p0_body(a_ref, gt_ref, eu0_ref, ei0t_ref,
             c_ref, a16_ref, raw1_ref, zu1_ref, zi1t_ref, reg_ref):
    a = a_ref[...]
    a16_ref[...] = a.astype(BF16)
    y = _dot_t1(a_ref[...], gt_ref[...])
    zu1_ref[...] = _dot_t1(a_ref[...], ei0t_ref[...])

    @pl.when(pl.program_id(0) == 0)
    def _():
        c_ref[...] = jnp.zeros_like(c_ref)
        raw1_ref[...] = jnp.zeros_like(raw1_ref)
        zi1t_ref[...] = jnp.zeros_like(zi1t_ref)
        reg_ref[...] = jnp.reshape(
            jnp.sum(ei0t_ref[...] * ei0t_ref[...]), (1, 1))

    c_ref[...] += _dot_t0(y, y)
    raw1_ref[...] += _dot_t0(y, a_ref[...])
    zi1t_ref[...] += _dot_t0(eu0_ref[...], a_ref[...])
    reg_ref[...] += jnp.reshape(jnp.sum(eu0_ref[...] * eu0_ref[...]), (1, 1))


def _pass_a(adj, gt, eu0, ei0t):
    kq = gt.shape[0]
    bm0 = 400
    return pl.pallas_call(
        _p0_body,
        grid=(N_U // bm0,),
        in_specs=[pl.BlockSpec((bm0, N_I), lambda i: (i, 0)),
                  pl.BlockSpec((kq, N_I), lambda i: (0, 0)),
                  pl.BlockSpec((bm0, DIM), lambda i: (i, 0)),
                  pl.BlockSpec((DIM, N_I), lambda i: (0, 0))],
        out_specs=[pl.BlockSpec((kq, kq), lambda i: (0, 0)),
                   pl.BlockSpec((bm0, N_I), lambda i: (i, 0)),
                   pl.BlockSpec((kq, N_I), lambda i: (0, 0)),
                   pl.BlockSpec((bm0, DIM), lambda i: (i, 0)),
                   pl.BlockSpec((DIM, N_I), lambda i: (0, 0)),
                   pl.BlockSpec((1, 1), lambda i: (0, 0))],
        out_shape=[jax.ShapeDtypeStruct((kq, kq), F32),
                   jax.ShapeDtypeStruct((N_U, N_I), BF16),
                   jax.ShapeDtypeStruct((kq, N_I), F32),
                   jax.ShapeDtypeStruct((N_U, DIM), F32),
                   jax.ShapeDtypeStruct((DIM, N_I), F32),
                   jax.ShapeDtypeStruct((1, 1), F32)],
    )(adj, gt, eu0, ei0t)


def _orth_chain(c_prev, raw):
    wt = _dot(_cholinv_math(c_prev), raw)
    x = _cholinv_math(_dot_t1(wt, wt))
    return _dot(x, wt)


def _pb_body(a_ref, c0_ref, raw1_ref, c_ref, raw2_ref, zt_ref):
    @pl.when(pl.program_id(0) == 0)
    def _():
        zt_ref[...] = _orth_chain(c0_ref[...], raw1_ref[...]).astype(BF16)
        c_ref[...] = jnp.zeros_like(c_ref)
        raw2_ref[...] = jnp.zeros_like(raw2_ref)

    y = _dot_t1(a_ref[...], zt_ref[...])
    c_ref[...] += _dot_t0(y, y)
    raw2_ref[...] += _dot_t0(y.astype(BF16), a_ref[...])


def _pass_b(a16, c0, raw1):
    kq = SVD_Q
    return pl.pallas_call(
        _pb_body,
        grid=(GRID_U,),
        in_specs=[pl.BlockSpec((BM, N_I), lambda i: (i, 0)),
                  pl.BlockSpec((kq, kq), lambda i: (0, 0)),
                  pl.BlockSpec((kq, N_I), lambda i: (0, 0))],
        out_specs=[pl.BlockSpec((kq, kq), lambda i: (0, 0)),
                   pl.BlockSpec((kq, N_I), lambda i: (0, 0))],
        out_shape=[jax.ShapeDtypeStruct((kq, kq), F32),
                   jax.ShapeDtypeStruct((kq, N_I), F32)],
        scratch_shapes=[pltpu.VMEM((kq, N_I), BF16)],
    )(a16, c0, raw1)


def _cholinv_math(C):
    q = SVD_Q
    ri = jax.lax.broadcasted_iota(jnp.int32, (q, q), 0)
    ci = jax.lax.broadcasted_iota(jnp.int32, (q, q), 1)
    ri1 = jax.lax.broadcasted_iota(jnp.int32, (q, 1), 0)
    ci1 = jax.lax.broadcasted_iota(jnp.int32, (1, q), 1)

    def chol_step(j, carry):
        L, Ck = carry
        dj = jnp.sum(jnp.where((ri == j) & (ci == j), Ck, 0.0))
        inv_s = jax.lax.rsqrt(dj)
        col = jnp.sum(jnp.where(ci == j, Ck, 0.0), axis=1, keepdims=True)
        col = jnp.where(ri1 >= j, col, 0.0) * inv_s
        row = jnp.sum(jnp.where(ri == j, Ck, 0.0), axis=0, keepdims=True)
        row = jnp.where(ci1 >= j, row, 0.0) * inv_s
        L = L + jnp.where(ci == j, col, 0.0)
        Ck = Ck - col * row
        return L, Ck

    L, _ = jax.lax.fori_loop(0, q, chol_step, (jnp.zeros_like(C), C))

    def inv_step(i, X):
        lrow = jnp.sum(jnp.where(ri == i, L, 0.0), axis=0, keepdims=True)
        dii = jnp.sum(jnp.where(ci1 == i, lrow, 0.0))
        lrow = jnp.where(ci1 < i, lrow, 0.0)
        prod = _dot(lrow, X, precision=_HI)
        xrow = (jnp.where(ci1 == i, 1.0, 0.0) - prod) / dii
        return X + jnp.where(ri == i, xrow, 0.0)

    return jax.lax.fori_loop(0, q, inv_step, jnp.zeros_like(C))


def _pc_body(a_ref, c1_ref, raw2_ref, zu1_ref, zi1t_ref, eu0_ref, ei0t_ref,
             y_ref, c_ref, bt_ref, eu_ref, eit_ref, t2t_ref, zt_ref):
    @pl.when(pl.program_id(0) == 0)
    def _():
        zt_ref[...] = _orth_chain(c1_ref[...], raw2_ref[...]).astype(BF16)
        c_ref[...] = jnp.zeros_like(c_ref)
        bt_ref[...] = jnp.zeros_like(bt_ref)
        eit_ref[...] = ei0t_ref[...] + zi1t_ref[...]
        t2t_ref[...] = jnp.zeros_like(t2t_ref)

    y = _dot_t1(a_ref[...], zt_ref[...])
    y_ref[...] = y
    zu1 = zu1_ref[...]
    eu0 = eu0_ref[...]
    eu_ref[...] = eu0 + zu1 + _dot_t1(a_ref[...], zi1t_ref[...].astype(BF16))

    c_ref[...] += _dot_t0(y, y)
    bt_ref[...] += _dot_t0(y.astype(BF16), a_ref[...])
    eit_ref[...] += _dot_t0(zu1.astype(BF16), a_ref[...])
    t2t_ref[...] += _dot_t0(eu0 + zu1, y)


def _pass_c(a16, c1, raw2, zu1, zi1t, eu0, ei0t):
    return pl.pallas_call(
        _pc_body,
        grid=(GRID_U,),
        in_specs=[pl.BlockSpec((BM, N_I), lambda i: (i, 0)),
                  pl.BlockSpec((SVD_Q, SVD_Q), lambda i: (0, 0)),
                  pl.BlockSpec((SVD_Q, N_I), lambda i: (0, 0)),
                  pl.BlockSpec((BM, DIM), lambda i: (i, 0)),
                  pl.BlockSpec((DIM, N_I), lambda i: (0, 0)),
                  pl.BlockSpec((BM, DIM), lambda i: (i, 0)),
                  pl.BlockSpec((DIM, N_I), lambda i: (0, 0))],
        out_specs=[pl.BlockSpec((BM, SVD_Q), lambda i: (i, 0)),
                   pl.BlockSpec((SVD_Q, SVD_Q), lambda i: (0, 0)),
                   pl.BlockSpec((SVD_Q, N_I), lambda i: (0, 0)),
                   pl.BlockSpec((BM, DIM), lambda i: (i, 0)),
                   pl.BlockSpec((DIM, N_I), lambda i: (0, 0)),
                   pl.BlockSpec((DIM, SVD_Q), lambda i: (0, 0))],
        out_shape=[jax.ShapeDtypeStruct((N_U, SVD_Q), F32),
                   jax.ShapeDtypeStruct((SVD_Q, SVD_Q), F32),
                   jax.ShapeDtypeStruct((SVD_Q, N_I), F32),
                   jax.ShapeDtypeStruct((N_U, DIM), F32),
                   jax.ShapeDtypeStruct((DIM, N_I), F32),
                   jax.ShapeDtypeStruct((DIM, SVD_Q), F32)],
        scratch_shapes=[pltpu.VMEM((SVD_Q, N_I), BF16)],
    )(a16, c1, raw2, zu1, zi1t, eu0, ei0t)


def _guei_body(y_ref, c_ref, bt_ref, t2t_ref, eu0_ref, eu_ref,
               ei0t_ref, zi1t_ref, eit_ref, ueu_ref, uei_ref, t1_ref):
    @pl.when(pl.program_id(0) == 0)
    def _():
        x = _cholinv_math(c_ref[...])
        m = _dot_t0(x, x)
        t1_ref[...] = _dot(
            m, _dot_t1(bt_ref[...], ei0t_ref[...] + zi1t_ref[...]))
        git = ei0t_ref[...] + _dot(_dot(t2t_ref[...], m), bt_ref[...])
        uei_ref[...] = jnp.concatenate([git, eit_ref[...]], axis=0)

    ueu_ref[...] = jnp.concatenate(
        [eu0_ref[...] + _dot(y_ref[...], t1_ref[...]), eu_ref[...]], axis=1)


def _guei(y2, c4, bt, t2t, eu0, e_u, ei0t, zi1t, eit):
    return pl.pallas_call(
        _guei_body,
        grid=(GRID_U,),
        in_specs=[pl.BlockSpec((BM, SVD_Q), lambda i: (i, 0)),
                  pl.BlockSpec((SVD_Q, SVD_Q), lambda i: (0, 0)),
                  pl.BlockSpec((SVD_Q, N_I), lambda i: (0, 0)),
                  pl.BlockSpec((DIM, SVD_Q), lambda i: (0, 0)),
                  pl.BlockSpec((BM, DIM), lambda i: (i, 0)),
                  pl.BlockSpec((BM, DIM), lambda i: (i, 0)),
                  pl.BlockSpec((DIM, N_I), lambda i: (0, 0)),
                  pl.BlockSpec((DIM, N_I), lambda i: (0, 0)),
                  pl.BlockSpec((DIM, N_I), lambda i: (0, 0))],
        out_specs=[pl.BlockSpec((BM, 2 * DIM), lambda i: (i, 0)),
                   pl.BlockSpec((2 * DIM, N_I), lambda i: (0, 0))],
        out_shape=[jax.ShapeDtypeStruct((N_U, 2 * DIM), F32),
                   jax.ShapeDtypeStruct((2 * DIM, N_I), F32)],
        scratch_shapes=[pltpu.VMEM((SVD_Q, DIM), F32)],
    )(y2, c4, bt, t2t, eu0, e_u, ei0t, zi1t, eit)


def _sc_gather_all(ue_u, ue_i, row_ids, col_ids, pos, neg):
    info = plsc.get_sparse_core_info()
    nc, ns = info.num_cores, info.num_subcores
    nw = nc * ns
    bpw = BATCH // nw
    mesh = plsc.VectorSubcoreMesh(core_axis_name="c", subcore_axis_name="s")
    out = jax.ShapeDtypeStruct((BATCH, 2 * DIM), F32)

    @functools.partial(
        pl.kernel, mesh=mesh,
        out_type=(out,) * 4,
        scratch_types=[pltpu.VMEM((bpw,), jnp.int32),
                       pltpu.VMEM((bpw, 2 * DIM), F32),
                       pltpu.SemaphoreType.DMA],
    )
    def k(tu_hbm, ti_hbm, rid_hbm, cid_hbm, pos_hbm, neg_hbm,
          o_ru, o_rc, o_rp, o_rn, idx_v, rows_v, sem):
        wid = lax.axis_index("s") * nc + lax.axis_index("c")
        base = wid * bpw

        def gather(idx_hbm, table_hbm, out_hbm):
            pltpu.sync_copy(idx_hbm.at[pl.ds(base, bpw)], idx_v)
            pltpu.async_copy(table_hbm.at[idx_v], rows_v, sem).wait()
            pltpu.sync_copy(rows_v, out_hbm.at[pl.ds(base, bpw)])

        gather(rid_hbm, tu_hbm, o_ru)
        gather(cid_hbm, ti_hbm, o_rc)
        gather(pos_hbm, ti_hbm, o_rp)
        gather(neg_hbm, ti_hbm, o_rn)

    return k(ue_u, ue_i, row_ids, col_ids, pos, neg)


_BB = 512
_NB = BATCH // _BB
_UCH = 2000
_ICH = 2500


def _loss_body(ueu_ref, eit_ref, reg_ref, ru_ref, rc_ref, rp_ref, rn_ref,
               loss_ref, oth_ref, acc_ref):
    i = pl.program_id(0)
    gur, eur = ru_ref[:, :DIM], ru_ref[:, DIM:]
    gic, eic = rc_ref[:, :DIM], rc_ref[:, DIM:]
    eip, ein = rp_ref[:, DIM:], rn_ref[:, DIM:]

    su = jnp.zeros((_BB, 1), F32)
    for kc in range(N_U // _UCH):
        logits = _dot_t1(gur, ueu_ref[kc * _UCH:(kc + 1) * _UCH, DIM:])
        su = su + jnp.sum(jnp.exp(logits * (1.0 / TEMP)), axis=1, keepdims=True)
    si = jnp.zeros((_BB, 1), F32)
    for kc in range(N_I // _ICH):
        logits = _dot(gic, eit_ref[:, kc * _ICH:(kc + 1) * _ICH])
        si = si + jnp.sum(jnp.exp(logits * (1.0 / TEMP)), axis=1, keepdims=True)
    nl_u = jnp.sum(jnp.log(su + 1e-08))
    nl_i = jnp.sum(jnp.log(si + 1e-08))

    pu = jnp.sum(gur * eur, axis=1, keepdims=True) * (1.0 / TEMP)
    pi = jnp.sum(gic * eic, axis=1, keepdims=True) * (1.0 / TEMP)
    pos = jnp.sum(jnp.clip(pu, -5.0, 5.0)) + jnp.sum(jnp.clip(pi, -5.0, 5.0))
    d = jnp.sum(eur * eip, axis=1, keepdims=True) - \
        jnp.sum(eur * ein, axis=1, keepdims=True)
    bpr = jnp.sum(jnp.log(1.0 + jnp.exp(-d)))

    upd = jnp.concatenate(
        [jnp.reshape(nl_u, (1, 1)), jnp.reshape(nl_i, (1, 1)),
         jnp.reshape(pos, (1, 1)), jnp.reshape(bpr, (1, 1)),
         jnp.zeros((1, 124), F32)], axis=1)

    @pl.when(i == 0)
    def _():
        acc_ref[...] = jnp.zeros_like(acc_ref)
    acc_ref[...] += upd

    @pl.when(i == _NB - 1)
    def _():
        acc = acc_ref[...]
        inv_b = 1.0 / BATCH
        neg_score = (acc[0, 0] + acc[0, 1]) * inv_b
        pos_score = acc[0, 2] * inv_b
        loss_bpr = acc[0, 3] * inv_b
        loss_cl = -pos_score + neg_score
        loss = loss_bpr + LAMBDA_1 * loss_cl + LAMBDA_2 * reg_ref[0, 0]
        loss_ref[...] = jnp.reshape(loss, (1, 1))
        oth_ref[...] = jnp.concatenate(
            [jnp.full((1, 1), loss_bpr, F32),
             jnp.full((1, 1), LAMBDA_1 * loss_cl, F32)], axis=1)


def _loss(ueu, eit, reg, ru, rc, rp, rn):
    bspec = pl.BlockSpec((_BB, 2 * DIM), lambda i: (i, 0))
    return pl.pallas_call(
        _loss_body,
        grid=(_NB,),
        in_specs=[pl.BlockSpec((N_U, 2 * DIM), lambda i: (0, 0)),
                  pl.BlockSpec((DIM, N_I), lambda i: (0, 0)),
                  pl.BlockSpec((1, 1), lambda i: (0, 0)),
                  bspec, bspec, bspec, bspec],
        out_specs=[pl.BlockSpec((1, 1), lambda i: (0, 0)),
                   pl.BlockSpec((1, 2), lambda i: (0, 0))],
        out_shape=[jax.ShapeDtypeStruct((1, 1), F32),
                   jax.ShapeDtypeStruct((1, 2), F32)],
        scratch_shapes=[pltpu.VMEM((1, 128), F32)],
    )(ueu, eit, reg, ru, rc, rp, rn)


def kernel(adj, row_ids, col_ids, pos, neg, E_u_0, E_i_0):
    g0t = jax.random.normal(jax.random.key(42), (N_I, SVD_Q), dtype=F32).T
    ei0t = E_i_0.T

    c0, a16, raw1, zu1, zi1t, reg = _pass_a(adj, g0t, E_u_0, ei0t)
    c1, raw2 = _pass_b(a16, c0, raw1)
    y2, c4, bt, e_u, e_it, t2t = _pass_c(a16, c1, raw2, zu1, zi1t,
                                         E_u_0, ei0t)
    ue_u, uei_t = _guei(y2, c4, bt, t2t, E_u_0, e_u, ei0t, zi1t, e_it)
    ue_i = uei_t.T

    ru, rc, rp, rn = _sc_gather_all(ue_u, ue_i, row_ids, col_ids, pos, neg)

    loss, oth = _loss(ue_u, e_it, reg, ru, rc, rp, rn)
    return loss[0, 0], oth[0]


# speedup vs baseline: 4.1274x; 1.0724x over previous
import functools

import jax
import jax.numpy as jnp
from jax import lax
from jax.experimental import pallas as pl
from jax.experimental.pallas import tpu as pltpu
from jax.experimental.pallas import tpu_sc as plsc

N_U = 10000
N_I = 5000
DIM = 64
TEMP = 0.2
LAMBDA_1 = 0.2
LAMBDA_2 = 1e-07
SVD_Q = 32
BATCH = 4096

BM = 1000
GRID_U = N_U // BM
F32 = jnp.float32
_HI = jax.lax.Precision.HIGHEST


def _dot(a, b, precision=None):
    return jax.lax.dot_general(a, b, (((1,), (0,)), ((), ())),
                               precision=precision, preferred_element_type=F32)


def _dot_t0(a, b):
    return jax.lax.dot_general(a, b, (((0,), (0,)), ((), ())),
                               preferred_element_type=F32)


def _dot_t1(a, b):
    return jax.lax.dot_general(a, b, (((1,), (1,)), ((), ())),
                               preferred_element_type=F32)


BF16 = jnp.bfloat16


def _p0_body(a_ref, gt_ref, eu0_ref, ei0t_ref,
             c_ref, raw1_ref, zu1_ref, zi1t_ref, reg_ref):
    y = _dot_t1(a_ref[...], gt_ref[...].astype(BF16))
    zu1_ref[...] = _dot_t1(a_ref[...], ei0t_ref[...].astype(BF16))

    @pl.when(pl.program_id(0) == 0)
    def _():
        c_ref[...] = jnp.zeros_like(c_ref)
        raw1_ref[...] = jnp.zeros_like(raw1_ref)
        zi1t_ref[...] = jnp.zeros_like(zi1t_ref)
        reg_ref[...] = jnp.reshape(
            jnp.sum(ei0t_ref[...] * ei0t_ref[...]), (1, 1))

    c_ref[...] += _dot_t0(y, y)
    raw1_ref[...] += _dot_t0(y.astype(BF16), a_ref[...])
    zi1t_ref[...] += _dot_t0(eu0_ref[...].astype(BF16), a_ref[...])
    reg_ref[...] += jnp.reshape(jnp.sum(eu0_ref[...] * eu0_ref[...]), (1, 1))


def _pass_a(a16, gt, eu0, ei0t):
    kq = gt.shape[0]
    return pl.pallas_call(
        _p0_body,
        grid=(GRID_U,),
        in_specs=[pl.BlockSpec((BM, N_I), lambda i: (i, 0)),
                  pl.BlockSpec((kq, N_I), lambda i: (0, 0)),
                  pl.BlockSpec((BM, DIM), lambda i: (i, 0)),
                  pl.BlockSpec((DIM, N_I), lambda i: (0, 0))],
        out_specs=[pl.BlockSpec((kq, kq), lambda i: (0, 0)),
                   pl.BlockSpec((kq, N_I), lambda i: (0, 0)),
                   pl.BlockSpec((BM, DIM), lambda i: (i, 0)),
                   pl.BlockSpec((DIM, N_I), lambda i: (0, 0)),
                   pl.BlockSpec((1, 1), lambda i: (0, 0))],
        out_shape=[jax.ShapeDtypeStruct((kq, kq), F32),
                   jax.ShapeDtypeStruct((kq, N_I), F32),
                   jax.ShapeDtypeStruct((N_U, DIM), F32),
                   jax.ShapeDtypeStruct((DIM, N_I), F32),
                   jax.ShapeDtypeStruct((1, 1), F32)],
    )(a16, gt, eu0, ei0t)


def _orth_chain(c_prev, raw):
    wt = _dot(_cholinv_math(c_prev), raw)
    x = _cholinv_math(_dot_t1(wt, wt))
    return _dot(x, wt)


def _pb_body(a_ref, c0_ref, raw1_ref, c_ref, raw2_ref, zt_ref):
    @pl.when(pl.program_id(0) == 0)
    def _():
        zt_ref[...] = _orth_chain(c0_ref[...], raw1_ref[...]).astype(BF16)
        c_ref[...] = jnp.zeros_like(c_ref)
        raw2_ref[...] = jnp.zeros_like(raw2_ref)

    y = _dot_t1(a_ref[...], zt_ref[...])
    c_ref[...] += _dot_t0(y, y)
    raw2_ref[...] += _dot_t0(y.astype(BF16), a_ref[...])


def _pass_b(a16, c0, raw1):
    kq = SVD_Q
    return pl.pallas_call(
        _pb_body,
        grid=(GRID_U,),
        in_specs=[pl.BlockSpec((BM, N_I), lambda i: (i, 0)),
                  pl.BlockSpec((kq, kq), lambda i: (0, 0)),
                  pl.BlockSpec((kq, N_I), lambda i: (0, 0))],
        out_specs=[pl.BlockSpec((kq, kq), lambda i: (0, 0)),
                   pl.BlockSpec((kq, N_I), lambda i: (0, 0))],
        out_shape=[jax.ShapeDtypeStruct((kq, kq), F32),
                   jax.ShapeDtypeStruct((kq, N_I), F32)],
        scratch_shapes=[pltpu.VMEM((kq, N_I), BF16)],
    )(a16, c0, raw1)


def _cholinv_math(C):
    q = SVD_Q
    ri = jax.lax.broadcasted_iota(jnp.int32, (q, q), 0)
    ci = jax.lax.broadcasted_iota(jnp.int32, (q, q), 1)
    ri1 = jax.lax.broadcasted_iota(jnp.int32, (q, 1), 0)
    ci1 = jax.lax.broadcasted_iota(jnp.int32, (1, q), 1)

    def chol_step(j, carry):
        L, Ck = carry
        dj = jnp.sum(jnp.where((ri == j) & (ci == j), Ck, 0.0))
        inv_s = jax.lax.rsqrt(dj)
        col = jnp.sum(jnp.where(ci == j, Ck, 0.0), axis=1, keepdims=True)
        col = jnp.where(ri1 >= j, col, 0.0) * inv_s
        row = jnp.sum(jnp.where(ri == j, Ck, 0.0), axis=0, keepdims=True)
        row = jnp.where(ci1 >= j, row, 0.0) * inv_s
        L = L + jnp.where(ci == j, col, 0.0)
        Ck = Ck - col * row
        return L, Ck

    L, _ = jax.lax.fori_loop(0, q, chol_step, (jnp.zeros_like(C), C))

    def inv_step(i, X):
        lrow = jnp.sum(jnp.where(ri == i, L, 0.0), axis=0, keepdims=True)
        dii = jnp.sum(jnp.where(ci1 == i, lrow, 0.0))
        lrow = jnp.where(ci1 < i, lrow, 0.0)
        prod = _dot(lrow, X, precision=_HI)
        xrow = (jnp.where(ci1 == i, 1.0, 0.0) - prod) / dii
        return X + jnp.where(ri == i, xrow, 0.0)

    return jax.lax.fori_loop(0, q, inv_step, jnp.zeros_like(C))


def _pc_body(a_ref, c1_ref, raw2_ref, zu1_ref, zi1t_ref, eu0_ref, ei0t_ref,
             y_ref, c_ref, bt_ref, eu_ref, eit_ref, t2t_ref, zt_ref):
    @pl.when(pl.program_id(0) == 0)
    def _():
        zt_ref[...] = _orth_chain(c1_ref[...], raw2_ref[...]).astype(BF16)
        c_ref[...] = jnp.zeros_like(c_ref)
        bt_ref[...] = jnp.zeros_like(bt_ref)
        eit_ref[...] = ei0t_ref[...] + zi1t_ref[...]
        t2t_ref[...] = jnp.zeros_like(t2t_ref)

    y = _dot_t1(a_ref[...], zt_ref[...])
    y_ref[...] = y
    zu1 = zu1_ref[...]
    eu0 = eu0_ref[...]
    eu_ref[...] = eu0 + zu1 + _dot_t1(a_ref[...], zi1t_ref[...].astype(BF16))

    c_ref[...] += _dot_t0(y, y)
    bt_ref[...] += _dot_t0(y.astype(BF16), a_ref[...])
    eit_ref[...] += _dot_t0(zu1.astype(BF16), a_ref[...])
    t2t_ref[...] += _dot_t0(eu0 + zu1, y)


def _pass_c(a16, c1, raw2, zu1, zi1t, eu0, ei0t):
    return pl.pallas_call(
        _pc_body,
        grid=(GRID_U,),
        in_specs=[pl.BlockSpec((BM, N_I), lambda i: (i, 0)),
                  pl.BlockSpec((SVD_Q, SVD_Q), lambda i: (0, 0)),
                  pl.BlockSpec((SVD_Q, N_I), lambda i: (0, 0)),
                  pl.BlockSpec((BM, DIM), lambda i: (i, 0)),
                  pl.BlockSpec((DIM, N_I), lambda i: (0, 0)),
                  pl.BlockSpec((BM, DIM), lambda i: (i, 0)),
                  pl.BlockSpec((DIM, N_I), lambda i: (0, 0))],
        out_specs=[pl.BlockSpec((BM, SVD_Q), lambda i: (i, 0)),
                   pl.BlockSpec((SVD_Q, SVD_Q), lambda i: (0, 0)),
                   pl.BlockSpec((SVD_Q, N_I), lambda i: (0, 0)),
                   pl.BlockSpec((BM, DIM), lambda i: (i, 0)),
                   pl.BlockSpec((DIM, N_I), lambda i: (0, 0)),
                   pl.BlockSpec((DIM, SVD_Q), lambda i: (0, 0))],
        out_shape=[jax.ShapeDtypeStruct((N_U, SVD_Q), F32),
                   jax.ShapeDtypeStruct((SVD_Q, SVD_Q), F32),
                   jax.ShapeDtypeStruct((SVD_Q, N_I), F32),
                   jax.ShapeDtypeStruct((N_U, DIM), F32),
                   jax.ShapeDtypeStruct((DIM, N_I), F32),
                   jax.ShapeDtypeStruct((DIM, SVD_Q), F32)],
        scratch_shapes=[pltpu.VMEM((SVD_Q, N_I), BF16)],
    )(a16, c1, raw2, zu1, zi1t, eu0, ei0t)


def _guei_body(y_ref, c_ref, bt_ref, t2t_ref, eu0_ref, eu_ref,
               ei0t_ref, zi1t_ref, eit_ref, ueu_ref, uei_ref, t1_ref):
    @pl.when(pl.program_id(0) == 0)
    def _():
        x = _cholinv_math(c_ref[...])
        m = _dot_t0(x, x)
        t1_ref[...] = _dot(
            m, _dot_t1(bt_ref[...], ei0t_ref[...] + zi1t_ref[...]))
        git = ei0t_ref[...] + _dot(_dot(t2t_ref[...], m), bt_ref[...])
        uei_ref[...] = jnp.concatenate([git, eit_ref[...]], axis=0)

    ueu_ref[...] = jnp.concatenate(
        [eu0_ref[...] + _dot(y_ref[...], t1_ref[...]), eu_ref[...]], axis=1)


def _guei(y2, c4, bt, t2t, eu0, e_u, ei0t, zi1t, eit):
    return pl.pallas_call(
        _guei_body,
        grid=(GRID_U,),
        in_specs=[pl.BlockSpec((BM, SVD_Q), lambda i: (i, 0)),
                  pl.BlockSpec((SVD_Q, SVD_Q), lambda i: (0, 0)),
                  pl.BlockSpec((SVD_Q, N_I), lambda i: (0, 0)),
                  pl.BlockSpec((DIM, SVD_Q), lambda i: (0, 0)),
                  pl.BlockSpec((BM, DIM), lambda i: (i, 0)),
                  pl.BlockSpec((BM, DIM), lambda i: (i, 0)),
                  pl.BlockSpec((DIM, N_I), lambda i: (0, 0)),
                  pl.BlockSpec((DIM, N_I), lambda i: (0, 0)),
                  pl.BlockSpec((DIM, N_I), lambda i: (0, 0))],
        out_specs=[pl.BlockSpec((BM, 2 * DIM), lambda i: (i, 0)),
                   pl.BlockSpec((2 * DIM, N_I), lambda i: (0, 0))],
        out_shape=[jax.ShapeDtypeStruct((N_U, 2 * DIM), F32),
                   jax.ShapeDtypeStruct((2 * DIM, N_I), F32)],
        scratch_shapes=[pltpu.VMEM((SVD_Q, DIM), F32)],
    )(y2, c4, bt, t2t, eu0, e_u, ei0t, zi1t, eit)


def _sc_gather_all(ue_u, ue_i, row_ids, col_ids, pos, neg):
    info = plsc.get_sparse_core_info()
    nc, ns = info.num_cores, info.num_subcores
    nw = nc * ns
    bpw = BATCH // nw
    mesh = plsc.VectorSubcoreMesh(core_axis_name="c", subcore_axis_name="s")
    out = jax.ShapeDtypeStruct((BATCH, 2 * DIM), F32)

    @functools.partial(
        pl.kernel, mesh=mesh,
        out_type=(out,) * 4,
        scratch_types=[pltpu.VMEM((bpw,), jnp.int32),
                       pltpu.VMEM((bpw, 2 * DIM), F32),
                       pltpu.SemaphoreType.DMA],
    )
    def k(tu_hbm, ti_hbm, rid_hbm, cid_hbm, pos_hbm, neg_hbm,
          o_ru, o_rc, o_rp, o_rn, idx_v, rows_v, sem):
        wid = lax.axis_index("s") * nc + lax.axis_index("c")
        base = wid * bpw

        def gather(idx_hbm, table_hbm, out_hbm):
            pltpu.sync_copy(idx_hbm.at[pl.ds(base, bpw)], idx_v)
            pltpu.async_copy(table_hbm.at[idx_v], rows_v, sem).wait()
            pltpu.sync_copy(rows_v, out_hbm.at[pl.ds(base, bpw)])

        gather(rid_hbm, tu_hbm, o_ru)
        gather(cid_hbm, ti_hbm, o_rc)
        gather(pos_hbm, ti_hbm, o_rp)
        gather(neg_hbm, ti_hbm, o_rn)

    return k(ue_u, ue_i, row_ids, col_ids, pos, neg)


_BB = 512
_NB = BATCH // _BB
_UCH = 2000
_ICH = 2500


def _loss_body(ueu_ref, eit_ref, reg_ref, ru_ref, rc_ref, rp_ref, rn_ref,
               loss_ref, oth_ref, acc_ref):
    i = pl.program_id(0)
    gur, eur = ru_ref[:, :DIM], ru_ref[:, DIM:]
    gic, eic = rc_ref[:, :DIM], rc_ref[:, DIM:]
    eip, ein = rp_ref[:, DIM:], rn_ref[:, DIM:]

    su = jnp.zeros((_BB, 1), F32)
    for kc in range(N_U // _UCH):
        logits = _dot_t1(gur, ueu_ref[kc * _UCH:(kc + 1) * _UCH, DIM:])
        su = su + jnp.sum(jnp.exp(logits * (1.0 / TEMP)), axis=1, keepdims=True)
    si = jnp.zeros((_BB, 1), F32)
    for kc in range(N_I // _ICH):
        logits = _dot(gic, eit_ref[:, kc * _ICH:(kc + 1) * _ICH])
        si = si + jnp.sum(jnp.exp(logits * (1.0 / TEMP)), axis=1, keepdims=True)
    nl_u = jnp.sum(jnp.log(su + 1e-08))
    nl_i = jnp.sum(jnp.log(si + 1e-08))

    pu = jnp.sum(gur * eur, axis=1, keepdims=True) * (1.0 / TEMP)
    pi = jnp.sum(gic * eic, axis=1, keepdims=True) * (1.0 / TEMP)
    pos = jnp.sum(jnp.clip(pu, -5.0, 5.0)) + jnp.sum(jnp.clip(pi, -5.0, 5.0))
    d = jnp.sum(eur * eip, axis=1, keepdims=True) - \
        jnp.sum(eur * ein, axis=1, keepdims=True)
    bpr = jnp.sum(jnp.log(1.0 + jnp.exp(-d)))

    upd = jnp.concatenate(
        [jnp.reshape(nl_u, (1, 1)), jnp.reshape(nl_i, (1, 1)),
         jnp.reshape(pos, (1, 1)), jnp.reshape(bpr, (1, 1)),
         jnp.zeros((1, 124), F32)], axis=1)

    @pl.when(i == 0)
    def _():
        acc_ref[...] = jnp.zeros_like(acc_ref)
    acc_ref[...] += upd

    @pl.when(i == _NB - 1)
    def _():
        acc = acc_ref[...]
        inv_b = 1.0 / BATCH
        neg_score = (acc[0, 0] + acc[0, 1]) * inv_b
        pos_score = acc[0, 2] * inv_b
        loss_bpr = acc[0, 3] * inv_b
        loss_cl = -pos_score + neg_score
        loss = loss_bpr + LAMBDA_1 * loss_cl + LAMBDA_2 * reg_ref[0, 0]
        loss_ref[...] = jnp.reshape(loss, (1, 1))
        oth_ref[...] = jnp.concatenate(
            [jnp.full((1, 1), loss_bpr, F32),
             jnp.full((1, 1), LAMBDA_1 * loss_cl, F32)], axis=1)


def _loss(ueu, eit, reg, ru, rc, rp, rn):
    bspec = pl.BlockSpec((_BB, 2 * DIM), lambda i: (i, 0))
    return pl.pallas_call(
        _loss_body,
        grid=(_NB,),
        in_specs=[pl.BlockSpec((N_U, 2 * DIM), lambda i: (0, 0)),
                  pl.BlockSpec((DIM, N_I), lambda i: (0, 0)),
                  pl.BlockSpec((1, 1), lambda i: (0, 0)),
                  bspec, bspec, bspec, bspec],
        out_specs=[pl.BlockSpec((1, 1), lambda i: (0, 0)),
                   pl.BlockSpec((1, 2), lambda i: (0, 0))],
        out_shape=[jax.ShapeDtypeStruct((1, 1), F32),
                   jax.ShapeDtypeStruct((1, 2), F32)],
        scratch_shapes=[pltpu.VMEM((1, 128), F32)],
    )(ueu, eit, reg, ru, rc, rp, rn)


def kernel(adj, row_ids, col_ids, pos, neg, E_u_0, E_i_0):
    g0t = jax.random.normal(jax.random.key(42), (N_I, SVD_Q), dtype=F32).T
    ei0t = E_i_0.T

    a16 = adj.astype(BF16)
    c0, raw1, zu1, zi1t, reg = _pass_a(a16, g0t, E_u_0, ei0t)
    c1, raw2 = _pass_b(a16, c0, raw1)
    y2, c4, bt, e_u, e_it, t2t = _pass_c(a16, c1, raw2, zu1, zi1t,
                                         E_u_0, ei0t)
    ue_u, uei_t = _guei(y2, c4, bt, t2t, E_u_0, e_u, ei0t, zi1t, e_it)
    ue_i = uei_t.T

    ru, rc, rp, rn = _sc_gather_all(ue_u, ue_i, row_ids, col_ids, pos, neg)

    loss, oth = _loss(ue_u, e_it, reg, ru, rc, rp, rn)
    return loss[0, 0], oth[0]


# speedup vs baseline: 4.2346x; 1.0260x over previous
import functools

import jax
import jax.numpy as jnp
from jax import lax
from jax.experimental import pallas as pl
from jax.experimental.pallas import tpu as pltpu
from jax.experimental.pallas import tpu_sc as plsc

N_U = 10000
N_I = 5000
DIM = 64
TEMP = 0.2
LAMBDA_1 = 0.2
LAMBDA_2 = 1e-07
SVD_Q = 32
BATCH = 4096

BM = 1000
GRID_U = N_U // BM
F32 = jnp.float32
_HI = jax.lax.Precision.HIGHEST


def _dot(a, b, precision=None):
    return jax.lax.dot_general(a, b, (((1,), (0,)), ((), ())),
                               precision=precision, preferred_element_type=F32)


def _dot_t0(a, b):
    return jax.lax.dot_general(a, b, (((0,), (0,)), ((), ())),
                               preferred_element_type=F32)


def _dot_t1(a, b):
    return jax.lax.dot_general(a, b, (((1,), (1,)), ((), ())),
                               preferred_element_type=F32)


BF16 = jnp.bfloat16


def _p0_body(a_ref, gt_ref, eu0_ref, ei0t_ref,
             c_ref, raw1_ref, zu1_ref, zi1t_ref, reg_ref):
    y = _dot_t1(a_ref[...], gt_ref[...].astype(BF16))
    zu1_ref[...] = _dot_t1(a_ref[...], ei0t_ref[...].astype(BF16))

    @pl.when(pl.program_id(0) == 0)
    def _():
        c_ref[...] = jnp.zeros_like(c_ref)
        raw1_ref[...] = jnp.zeros_like(raw1_ref)
        zi1t_ref[...] = jnp.zeros_like(zi1t_ref)
        reg_ref[...] = jnp.reshape(
            jnp.sum(ei0t_ref[...] * ei0t_ref[...]), (1, 1))

    c_ref[...] += _dot_t0(y, y)
    raw1_ref[...] += _dot_t0(y.astype(BF16), a_ref[...])
    zi1t_ref[...] += _dot_t0(eu0_ref[...].astype(BF16), a_ref[...])
    reg_ref[...] += jnp.reshape(jnp.sum(eu0_ref[...] * eu0_ref[...]), (1, 1))


def _pass_a(a16, gt, eu0, ei0t):
    kq = gt.shape[0]
    return pl.pallas_call(
        _p0_body,
        grid=(GRID_U,),
        in_specs=[pl.BlockSpec((BM, N_I), lambda i: (i, 0)),
                  pl.BlockSpec((kq, N_I), lambda i: (0, 0)),
                  pl.BlockSpec((BM, DIM), lambda i: (i, 0)),
                  pl.BlockSpec((DIM, N_I), lambda i: (0, 0))],
        out_specs=[pl.BlockSpec((kq, kq), lambda i: (0, 0)),
                   pl.BlockSpec((kq, N_I), lambda i: (0, 0)),
                   pl.BlockSpec((BM, DIM), lambda i: (i, 0)),
                   pl.BlockSpec((DIM, N_I), lambda i: (0, 0)),
                   pl.BlockSpec((1, 1), lambda i: (0, 0))],
        out_shape=[jax.ShapeDtypeStruct((kq, kq), F32),
                   jax.ShapeDtypeStruct((kq, N_I), F32),
                   jax.ShapeDtypeStruct((N_U, DIM), F32),
                   jax.ShapeDtypeStruct((DIM, N_I), F32),
                   jax.ShapeDtypeStruct((1, 1), F32)],
    )(a16, gt, eu0, ei0t)


def _eye(q):
    ri = jax.lax.broadcasted_iota(jnp.int32, (q, q), 0)
    ci = jax.lax.broadcasted_iota(jnp.int32, (q, q), 1)
    return jnp.where(ri == ci, 1.0, 0.0).astype(F32)


def _trace(C):
    q = C.shape[0]
    ri = jax.lax.broadcasted_iota(jnp.int32, (q, q), 0)
    ci = jax.lax.broadcasted_iota(jnp.int32, (q, q), 1)
    return jnp.sum(jnp.where(ri == ci, C, 0.0))


def _ns_invsqrt(C, iters=20):
    eye = _eye(C.shape[0])
    s = _trace(C)
    y = C * (1.0 / s)
    z = eye
    for _ in range(iters):
        t = 1.5 * eye - 0.5 * _dot(z, y, precision=_HI)
        y = _dot(y, t, precision=_HI)
        z = _dot(t, z, precision=_HI)
    return z * jax.lax.rsqrt(s)


def _ns_inv(C, iters=20):
    eye = _eye(C.shape[0])
    x = eye * (1.0 / _trace(C))
    for _ in range(iters):
        x = _dot(x, 2.0 * eye - _dot(C, x, precision=_HI), precision=_HI)
    return x


def _orth_chain(c_prev, raw):
    wt = _dot(_ns_invsqrt(c_prev), raw, precision=_HI)
    s = _ns_invsqrt(_dot_t1(wt, wt))
    return _dot(s, wt, precision=_HI)


def _pb_body(a_ref, c0_ref, raw1_ref, c_ref, raw2_ref, zt_ref):
    @pl.when(pl.program_id(0) == 0)
    def _():
        zt_ref[...] = _orth_chain(c0_ref[...], raw1_ref[...]).astype(BF16)
        c_ref[...] = jnp.zeros_like(c_ref)
        raw2_ref[...] = jnp.zeros_like(raw2_ref)

    y = _dot_t1(a_ref[...], zt_ref[...])
    c_ref[...] += _dot_t0(y, y)
    raw2_ref[...] += _dot_t0(y.astype(BF16), a_ref[...])


def _pass_b(a16, c0, raw1):
    kq = SVD_Q
    return pl.pallas_call(
        _pb_body,
        grid=(GRID_U,),
        in_specs=[pl.BlockSpec((BM, N_I), lambda i: (i, 0)),
                  pl.BlockSpec((kq, kq), lambda i: (0, 0)),
                  pl.BlockSpec((kq, N_I), lambda i: (0, 0))],
        out_specs=[pl.BlockSpec((kq, kq), lambda i: (0, 0)),
                   pl.BlockSpec((kq, N_I), lambda i: (0, 0))],
        out_shape=[jax.ShapeDtypeStruct((kq, kq), F32),
                   jax.ShapeDtypeStruct((kq, N_I), F32)],
        scratch_shapes=[pltpu.VMEM((kq, N_I), BF16)],
    )(a16, c0, raw1)


def _pc_body(a_ref, c1_ref, raw2_ref, zu1_ref, zi1t_ref, eu0_ref, ei0t_ref,
             y_ref, c_ref, bt_ref, eu_ref, eit_ref, t2t_ref, zt_ref):
    @pl.when(pl.program_id(0) == 0)
    def _():
        zt_ref[...] = _orth_chain(c1_ref[...], raw2_ref[...]).astype(BF16)
        c_ref[...] = jnp.zeros_like(c_ref)
        bt_ref[...] = jnp.zeros_like(bt_ref)
        eit_ref[...] = ei0t_ref[...] + zi1t_ref[...]
        t2t_ref[...] = jnp.zeros_like(t2t_ref)

    y = _dot_t1(a_ref[...], zt_ref[...])
    y_ref[...] = y
    zu1 = zu1_ref[...]
    eu0 = eu0_ref[...]
    eu_ref[...] = eu0 + zu1 + _dot_t1(a_ref[...], zi1t_ref[...].astype(BF16))

    c_ref[...] += _dot_t0(y, y)
    bt_ref[...] += _dot_t0(y.astype(BF16), a_ref[...])
    eit_ref[...] += _dot_t0(zu1.astype(BF16), a_ref[...])
    t2t_ref[...] += _dot_t0(eu0 + zu1, y)


def _pass_c(a16, c1, raw2, zu1, zi1t, eu0, ei0t):
    return pl.pallas_call(
        _pc_body,
        grid=(GRID_U,),
        in_specs=[pl.BlockSpec((BM, N_I), lambda i: (i, 0)),
                  pl.BlockSpec((SVD_Q, SVD_Q), lambda i: (0, 0)),
                  pl.BlockSpec((SVD_Q, N_I), lambda i: (0, 0)),
                  pl.BlockSpec((BM, DIM), lambda i: (i, 0)),
                  pl.BlockSpec((DIM, N_I), lambda i: (0, 0)),
                  pl.BlockSpec((BM, DIM), lambda i: (i, 0)),
                  pl.BlockSpec((DIM, N_I), lambda i: (0, 0))],
        out_specs=[pl.BlockSpec((BM, SVD_Q), lambda i: (i, 0)),
                   pl.BlockSpec((SVD_Q, SVD_Q), lambda i: (0, 0)),
                   pl.BlockSpec((SVD_Q, N_I), lambda i: (0, 0)),
                   pl.BlockSpec((BM, DIM), lambda i: (i, 0)),
                   pl.BlockSpec((DIM, N_I), lambda i: (0, 0)),
                   pl.BlockSpec((DIM, SVD_Q), lambda i: (0, 0))],
        out_shape=[jax.ShapeDtypeStruct((N_U, SVD_Q), F32),
                   jax.ShapeDtypeStruct((SVD_Q, SVD_Q), F32),
                   jax.ShapeDtypeStruct((SVD_Q, N_I), F32),
                   jax.ShapeDtypeStruct((N_U, DIM), F32),
                   jax.ShapeDtypeStruct((DIM, N_I), F32),
                   jax.ShapeDtypeStruct((DIM, SVD_Q), F32)],
        scratch_shapes=[pltpu.VMEM((SVD_Q, N_I), BF16)],
    )(a16, c1, raw2, zu1, zi1t, eu0, ei0t)


def _guei_body(y_ref, c_ref, bt_ref, t2t_ref, eu0_ref, eu_ref,
               ei0t_ref, zi1t_ref, eit_ref, ueu_ref, uei_ref, t1_ref):
    @pl.when(pl.program_id(0) == 0)
    def _():
        m = _ns_inv(c_ref[...])
        t1_ref[...] = _dot(
            m, _dot_t1(bt_ref[...], ei0t_ref[...] + zi1t_ref[...]))
        git = ei0t_ref[...] + _dot(_dot(t2t_ref[...], m), bt_ref[...])
        uei_ref[...] = jnp.concatenate([git, eit_ref[...]], axis=0)

    ueu_ref[...] = jnp.concatenate(
        [eu0_ref[...] + _dot(y_ref[...], t1_ref[...]), eu_ref[...]], axis=1)


def _guei(y2, c4, bt, t2t, eu0, e_u, ei0t, zi1t, eit):
    return pl.pallas_call(
        _guei_body,
        grid=(GRID_U,),
        in_specs=[pl.BlockSpec((BM, SVD_Q), lambda i: (i, 0)),
                  pl.BlockSpec((SVD_Q, SVD_Q), lambda i: (0, 0)),
                  pl.BlockSpec((SVD_Q, N_I), lambda i: (0, 0)),
                  pl.BlockSpec((DIM, SVD_Q), lambda i: (0, 0)),
                  pl.BlockSpec((BM, DIM), lambda i: (i, 0)),
                  pl.BlockSpec((BM, DIM), lambda i: (i, 0)),
                  pl.BlockSpec((DIM, N_I), lambda i: (0, 0)),
                  pl.BlockSpec((DIM, N_I), lambda i: (0, 0)),
                  pl.BlockSpec((DIM, N_I), lambda i: (0, 0))],
        out_specs=[pl.BlockSpec((BM, 2 * DIM), lambda i: (i, 0)),
                   pl.BlockSpec((2 * DIM, N_I), lambda i: (0, 0))],
        out_shape=[jax.ShapeDtypeStruct((N_U, 2 * DIM), F32),
                   jax.ShapeDtypeStruct((2 * DIM, N_I), F32)],
        scratch_shapes=[pltpu.VMEM((SVD_Q, DIM), F32)],
    )(y2, c4, bt, t2t, eu0, e_u, ei0t, zi1t, eit)


def _sc_gather_all(ue_u, ue_i, row_ids, col_ids, pos, neg):
    info = plsc.get_sparse_core_info()
    nc, ns = info.num_cores, info.num_subcores
    nw = nc * ns
    bpw = BATCH // nw
    mesh = plsc.VectorSubcoreMesh(core_axis_name="c", subcore_axis_name="s")
    out = jax.ShapeDtypeStruct((BATCH, 2 * DIM), F32)

    @functools.partial(
        pl.kernel, mesh=mesh,
        out_type=(out,) * 4,
        scratch_types=[pltpu.VMEM((bpw,), jnp.int32),
                       pltpu.VMEM((bpw, 2 * DIM), F32),
                       pltpu.SemaphoreType.DMA],
    )
    def k(tu_hbm, ti_hbm, rid_hbm, cid_hbm, pos_hbm, neg_hbm,
          o_ru, o_rc, o_rp, o_rn, idx_v, rows_v, sem):
        wid = lax.axis_index("s") * nc + lax.axis_index("c")
        base = wid * bpw

        def gather(idx_hbm, table_hbm, out_hbm):
            pltpu.sync_copy(idx_hbm.at[pl.ds(base, bpw)], idx_v)
            pltpu.async_copy(table_hbm.at[idx_v], rows_v, sem).wait()
            pltpu.sync_copy(rows_v, out_hbm.at[pl.ds(base, bpw)])

        gather(rid_hbm, tu_hbm, o_ru)
        gather(cid_hbm, ti_hbm, o_rc)
        gather(pos_hbm, ti_hbm, o_rp)
        gather(neg_hbm, ti_hbm, o_rn)

    return k(ue_u, ue_i, row_ids, col_ids, pos, neg)


_BB = 512
_NB = BATCH // _BB
_UCH = 2000
_ICH = 2500


def _loss_body(ueu_ref, eit_ref, reg_ref, ru_ref, rc_ref, rp_ref, rn_ref,
               loss_ref, oth_ref, acc_ref):
    i = pl.program_id(0)
    gur, eur = ru_ref[:, :DIM], ru_ref[:, DIM:]
    gic, eic = rc_ref[:, :DIM], rc_ref[:, DIM:]
    eip, ein = rp_ref[:, DIM:], rn_ref[:, DIM:]

    su = jnp.zeros((_BB, 1), F32)
    for kc in range(N_U // _UCH):
        logits = _dot_t1(gur, ueu_ref[kc * _UCH:(kc + 1) * _UCH, DIM:])
        su = su + jnp.sum(jnp.exp(logits * (1.0 / TEMP)), axis=1, keepdims=True)
    si = jnp.zeros((_BB, 1), F32)
    for kc in range(N_I // _ICH):
        logits = _dot(gic, eit_ref[:, kc * _ICH:(kc + 1) * _ICH])
        si = si + jnp.sum(jnp.exp(logits * (1.0 / TEMP)), axis=1, keepdims=True)
    nl_u = jnp.sum(jnp.log(su + 1e-08))
    nl_i = jnp.sum(jnp.log(si + 1e-08))

    pu = jnp.sum(gur * eur, axis=1, keepdims=True) * (1.0 / TEMP)
    pi = jnp.sum(gic * eic, axis=1, keepdims=True) * (1.0 / TEMP)
    pos = jnp.sum(jnp.clip(pu, -5.0, 5.0)) + jnp.sum(jnp.clip(pi, -5.0, 5.0))
    d = jnp.sum(eur * eip, axis=1, keepdims=True) - \
        jnp.sum(eur * ein, axis=1, keepdims=True)
    bpr = jnp.sum(jnp.log(1.0 + jnp.exp(-d)))

    upd = jnp.concatenate(
        [jnp.reshape(nl_u, (1, 1)), jnp.reshape(nl_i, (1, 1)),
         jnp.reshape(pos, (1, 1)), jnp.reshape(bpr, (1, 1)),
         jnp.zeros((1, 124), F32)], axis=1)

    @pl.when(i == 0)
    def _():
        acc_ref[...] = jnp.zeros_like(acc_ref)
    acc_ref[...] += upd

    @pl.when(i == _NB - 1)
    def _():
        acc = acc_ref[...]
        inv_b = 1.0 / BATCH
        neg_score = (acc[0, 0] + acc[0, 1]) * inv_b
        pos_score = acc[0, 2] * inv_b
        loss_bpr = acc[0, 3] * inv_b
        loss_cl = -pos_score + neg_score
        loss = loss_bpr + LAMBDA_1 * loss_cl + LAMBDA_2 * reg_ref[0, 0]
        loss_ref[...] = jnp.reshape(loss, (1, 1))
        oth_ref[...] = jnp.concatenate(
            [jnp.full((1, 1), loss_bpr, F32),
             jnp.full((1, 1), LAMBDA_1 * loss_cl, F32)], axis=1)


def _loss(ueu, eit, reg, ru, rc, rp, rn):
    bspec = pl.BlockSpec((_BB, 2 * DIM), lambda i: (i, 0))
    return pl.pallas_call(
        _loss_body,
        grid=(_NB,),
        in_specs=[pl.BlockSpec((N_U, 2 * DIM), lambda i: (0, 0)),
                  pl.BlockSpec((DIM, N_I), lambda i: (0, 0)),
                  pl.BlockSpec((1, 1), lambda i: (0, 0)),
                  bspec, bspec, bspec, bspec],
        out_specs=[pl.BlockSpec((1, 1), lambda i: (0, 0)),
                   pl.BlockSpec((1, 2), lambda i: (0, 0))],
        out_shape=[jax.ShapeDtypeStruct((1, 1), F32),
                   jax.ShapeDtypeStruct((1, 2), F32)],
        scratch_shapes=[pltpu.VMEM((1, 128), F32)],
    )(ueu, eit, reg, ru, rc, rp, rn)


def kernel(adj, row_ids, col_ids, pos, neg, E_u_0, E_i_0):
    g0t = jax.random.normal(jax.random.key(42), (N_I, SVD_Q), dtype=F32).T
    ei0t = E_i_0.T

    a16 = adj.astype(BF16)
    c0, raw1, zu1, zi1t, reg = _pass_a(a16, g0t, E_u_0, ei0t)
    c1, raw2 = _pass_b(a16, c0, raw1)
    y2, c4, bt, e_u, e_it, t2t = _pass_c(a16, c1, raw2, zu1, zi1t,
                                         E_u_0, ei0t)
    ue_u, uei_t = _guei(y2, c4, bt, t2t, E_u_0, e_u, ei0t, zi1t, e_it)
    ue_i = uei_t.T

    ru, rc, rp, rn = _sc_gather_all(ue_u, ue_i, row_ids, col_ids, pos, neg)

    loss, oth = _loss(ue_u, e_it, reg, ru, rc, rp, rn)
    return loss[0, 0], oth[0]
